# Initial kernel scaffold; baseline (speedup 1.0000x reference)
import functools

import jax
import jax.numpy as jnp
from jax import lax
from jax.experimental import pallas as pl
from jax.experimental.pallas import tpu as pltpu
from jax.experimental.pallas import tpu_sc as plsc

D = 128
N, E, G = 10000, 320000, 8
NP, EP = 10240, 327680
NC, NS = 2, 16
NW = NC * NS
EPW = EP // NW
BLKN = 1024
BLKE = 2048
F32 = jnp.float32


def _preproj_body(x_ref, b_ref, u_ref, wa_ref, wb_ref, wn_ref, wud_ref,
                  be1_ref, wun_ref,
                  xa_ref, xb_ref, xn_ref, ue_ref, un_ref, starts_ref):
    i = pl.program_id(0)
    x = x_ref[...]
    xa_ref[...] = jnp.dot(x, wa_ref[...], preferred_element_type=F32)
    xb_ref[...] = jnp.dot(x, wb_ref[...], preferred_element_type=F32)
    xn_ref[...] = jnp.dot(x, wn_ref[...], preferred_element_type=F32)

    @pl.when(i == 0)
    def _():
        u = u_ref[...]
        ue_ref[...] = jnp.dot(u, wud_ref[...], preferred_element_type=F32) + be1_ref[...]
        un_ref[...] = jnp.dot(u, wun_ref[...], preferred_element_type=F32)
        starts_ref[...] = jnp.zeros_like(starts_ref)

    b = b_ref[...]
    g = lax.broadcasted_iota(jnp.int32, (1, 128), 1)
    cmp = (b < g).astype(jnp.int32)
    s = jnp.sum(cmp, axis=0, keepdims=True)
    starts_ref[...] += jnp.broadcast_to(s, (8, 128))


def _preproj(x, batch_col, u, wa, wb, wn, wud, be1, wun):
    grid = (NP // BLKN,)
    full = lambda shp: pl.BlockSpec(shp, lambda i: (0, 0))
    return pl.pallas_call(
        _preproj_body,
        grid=grid,
        in_specs=[
            pl.BlockSpec((BLKN, D), lambda i: (i, 0)),
            pl.BlockSpec((BLKN, 1), lambda i: (i, 0)),
            full((G, D)), full((D, D)), full((D, D)), full((D, D)),
            full((D, D)), full((1, D)), full((D, D)),
        ],
        out_specs=[
            pl.BlockSpec((BLKN, D), lambda i: (i, 0)),
            pl.BlockSpec((BLKN, D), lambda i: (i, 0)),
            pl.BlockSpec((BLKN, D), lambda i: (i, 0)),
            full((G, D)), full((G, D)), full((8, 128)),
        ],
        out_shape=[
            jax.ShapeDtypeStruct((NP, D), F32),
            jax.ShapeDtypeStruct((NP, D), F32),
            jax.ShapeDtypeStruct((NP, D), F32),
            jax.ShapeDtypeStruct((G, D), F32),
            jax.ShapeDtypeStruct((G, D), F32),
            jax.ShapeDtypeStruct((8, 128), jnp.int32),
        ],
    )(x, batch_col, u, wa, wb, wn, wud, be1, wun)


def _sc_gather_body(xa_hbm, xb_hbm, row2d, col2d, ident_hbm, pre_hbm,
                    idxr, idxc, identv, bufa, bufb, sem):
    s = lax.axis_index("s")
    c = lax.axis_index("c")
    wid = c * NS + s
    base = wid * EPW
    pltpu.sync_copy(ident_hbm, identv)

    def body(k, carry):
        rb = base // 128 + k * 2
        pltpu.sync_copy(row2d.at[pl.ds(rb, 2)], idxr)
        pltpu.sync_copy(col2d.at[pl.ds(rb, 2)], idxc)
        cps = []
        for j in range(2):
            cps.append(pltpu.async_copy(
                xa_hbm.at[idxr.at[j]], bufa.at[pl.ds(j * 128, 128)], sem))
            cps.append(pltpu.async_copy(
                xb_hbm.at[idxc.at[j]], bufb.at[pl.ds(j * 128, 128)], sem))
        for cp in cps:
            cp.wait()
        for j in range(2):
            pltpu.sync_copy(bufb.at[pl.ds(j * 128, 128)],
                            bufa.at[identv.at[j]], add=True)
        pltpu.sync_copy(bufa, pre_hbm.at[pl.ds(base + k * 256, 256)])
        return carry

    lax.fori_loop(0, EPW // 256, body, 0)


def _sc_gather(xa, xb, row2d, col2d, ident):
    mesh = plsc.VectorSubcoreMesh(core_axis_name="c", subcore_axis_name="s")
    f = pl.kernel(
        _sc_gather_body,
        out_type=jax.ShapeDtypeStruct((EP, D), F32),
        mesh=mesh,
        scratch_types=[
            pltpu.VMEM((2, 128), jnp.int32),
            pltpu.VMEM((2, 128), jnp.int32),
            pltpu.VMEM((2, 128), jnp.int32),
            pltpu.VMEM((256, D), F32),
            pltpu.VMEM((256, D), F32),
            pltpu.SemaphoreType.DMA,
        ],
    )
    return f(xa, xb, row2d, col2d, ident)


def _edge_body(pre_ref, e_ref, r_ref, s_ref, ue_ref, w1c_ref, w2_ref, be2_ref,
               out_ref, sue_ref, cne_ref):
    i = pl.program_id(0)
    r = r_ref[...]
    gi = i * BLKE + lax.broadcasted_iota(jnp.int32, (BLKE, 1), 0)
    valid = (gi < E).astype(F32)
    bi = jnp.zeros((BLKE, 1), jnp.int32)
    for gg in range(1, 8):
        bi = bi + (r >= s_ref[gg:gg + 1, 0:1]).astype(jnp.int32)
    onehot = (bi == lax.broadcasted_iota(jnp.int32, (1, 8), 1)).astype(F32)
    onehot = onehot * valid
    uterm = jnp.dot(onehot, ue_ref[...], preferred_element_type=F32)
    h = jnp.maximum(pre_ref[...] +
                    jnp.dot(e_ref[...], w1c_ref[...], preferred_element_type=F32) +
                    uterm, 0.0)
    e_new = jnp.dot(h, w2_ref[...], preferred_element_type=F32) + be2_ref[...]
    out_ref[...] = e_new

    @pl.when(i == 0)
    def _():
        sue_ref[...] = jnp.zeros_like(sue_ref)
        cne_ref[...] = jnp.zeros_like(cne_ref)
    sue_ref[...] += lax.dot_general(onehot, e_new, (((0,), (0,)), ((), ())),
                                    preferred_element_type=F32)
    ones = jnp.ones((BLKE, 128), F32)
    cne_ref[...] += lax.dot_general(onehot, ones, (((0,), (0,)), ((), ())),
                                    preferred_element_type=F32)


def _edge_mlp(pre, e_pad, row_col, starts, ue, w1c, w2, be2):
    grid = (EP // BLKE,)
    full = lambda shp: pl.BlockSpec(shp, lambda i: (0, 0))
    return pl.pallas_call(
        _edge_body,
        grid=grid,
        in_specs=[
            pl.BlockSpec((BLKE, D), lambda i: (i, 0)),
            pl.BlockSpec((BLKE, D), lambda i: (i, 0)),
            pl.BlockSpec((BLKE, 1), lambda i: (i, 0)),
            full((8, 128)), full((G, D)), full((D, D)), full((D, D)),
            full((1, D)),
        ],
        out_specs=[
            pl.BlockSpec((BLKE, D), lambda i: (i, 0)),
            full((G, D)), full((G, D)),
        ],
        out_shape=[
            jax.ShapeDtypeStruct((EP, D), F32),
            jax.ShapeDtypeStruct((G, D), F32),
            jax.ShapeDtypeStruct((G, D), F32),
        ],
    )(pre, e_pad, row_col, starts, ue, w1c, w2, be2)


def _sc_scatter_body(enew, col2d, zeros_hbm, zcol_hbm, ones_hbm,
                     aggp, cntp,
                     buf, idxc, cbuf, onesv, aggS, cntS):
    s = lax.axis_index("s")
    c = lax.axis_index("c")
    wid = c * NS + s
    rows = NP // NS
    pltpu.sync_copy(ones_hbm, onesv)

    for t in range(rows // 128):
        pltpu.sync_copy(zeros_hbm.at[pl.ds(t * 128, 128)], buf)
        pltpu.sync_copy(buf, aggS.at[pl.ds(s * rows + t * 128, 128)])
    pltpu.sync_copy(zcol_hbm, cbuf)
    pltpu.sync_copy(cbuf, cntS.at[pl.ds(s * rows, rows)])
    plsc.subcore_barrier()

    base128 = wid * (EPW // 128)

    def body(k, carry):
        blk = base128 + k
        pltpu.sync_copy(enew.at[pl.ds(blk * 128, 128)], buf)
        pltpu.sync_copy(col2d.at[pl.ds(blk, 1)], idxc)
        pltpu.sync_copy(buf, aggS.at[idxc.at[0]], add=True)
        pltpu.sync_copy(onesv, cntS.at[idxc.at[0]], add=True)
        return carry

    lax.fori_loop(0, EPW // 128, body, 0)
    plsc.subcore_barrier()

    for t in range(rows // 128):
        r = s * rows + t * 128
        pltpu.sync_copy(aggS.at[pl.ds(r, 128)], buf)
        pltpu.sync_copy(buf, aggp.at[pl.ds(c * NP + r, 128)])
    pltpu.sync_copy(cntS.at[pl.ds(s * rows, rows)], cbuf)
    pltpu.sync_copy(cbuf, cntp.at[pl.ds(c * NP + s * rows, rows)])


def _sc_scatter(e_new, col2d, zeros_blk, zcol, onesb):
    mesh = plsc.VectorSubcoreMesh(core_axis_name="c", subcore_axis_name="s")
    f = pl.kernel(
        _sc_scatter_body,
        out_type=[
            jax.ShapeDtypeStruct((NC * NP, D), F32),
            jax.ShapeDtypeStruct((NC * NP, 1), F32),
        ],
        mesh=mesh,
        scratch_types=[
            pltpu.VMEM((128, D), F32),
            pltpu.VMEM((1, 128), jnp.int32),
            pltpu.VMEM((NP // NS, 1), F32),
            pltpu.VMEM((128, 1), F32),
            pltpu.VMEM_SHARED((NP, D), F32),
            pltpu.VMEM_SHARED((NP, 1), F32),
        ],
    )
    return f(e_new, col2d, zeros_blk, zcol, onesb)


def _node_body(xn_ref, p0_ref, p1_ref, c0_ref, c1_ref, b_ref, un_ref,
               u_ref, sue_ref, cne_ref, w1b_ref, bn1_ref, wn2_ref, bn2_ref,
               g1u_ref, g1x_ref, g1e_ref, bg1_ref, wg2_ref, bg2_ref,
               out_ref, uout_ref, sx_ref, cn_ref):
    i = pl.program_id(0)

    @pl.when(i == 0)
    def _():
        sx_ref[...] = jnp.zeros_like(sx_ref)
        cn_ref[...] = jnp.zeros_like(cn_ref)

    cnt = jnp.maximum(c0_ref[...] + c1_ref[...], 1.0)
    agg = (p0_ref[...] + p1_ref[...]) / cnt
    b = b_ref[...]
    onehot = (b == lax.broadcasted_iota(jnp.int32, (1, 8), 1)).astype(F32)
    z = (xn_ref[...] +
         jnp.dot(agg, w1b_ref[...], preferred_element_type=F32) +
         jnp.dot(onehot, un_ref[...], preferred_element_type=F32) +
         bn1_ref[...])
    h = jnp.maximum(z, 0.0)
    x_new = jnp.dot(h, wn2_ref[...], preferred_element_type=F32) + bn2_ref[...]
    out_ref[...] = x_new
    sx_ref[...] += lax.dot_general(onehot, x_new, (((0,), (0,)), ((), ())),
                                   preferred_element_type=F32)
    ones = jnp.ones((BLKN, 128), F32)
    cn_ref[...] += lax.dot_general(onehot, ones, (((0,), (0,)), ((), ())),
                                   preferred_element_type=F32)

    @pl.when(i == pl.num_programs(0) - 1)
    def _():
        sxm = sx_ref[...] / jnp.maximum(cn_ref[...], 1.0)
        sem = sue_ref[...] / jnp.maximum(cne_ref[...], 1.0)
        zg = (jnp.dot(u_ref[...], g1u_ref[...], preferred_element_type=F32) +
              jnp.dot(sxm, g1x_ref[...], preferred_element_type=F32) +
              jnp.dot(sem, g1e_ref[...], preferred_element_type=F32) +
              bg1_ref[...])
        hg = jnp.maximum(zg, 0.0)
        uout_ref[...] = jnp.dot(hg, wg2_ref[...], preferred_element_type=F32) + bg2_ref[...]


def _node_global(xn, p0, p1, c0, c1, batch_col, un, u, sue, cne,
                 w1b, bn1, wn2, bn2, g1u, g1x, g1e, bg1, wg2, bg2):
    grid = (NP // BLKN,)
    full = lambda shp: pl.BlockSpec(shp, lambda i: (0, 0))
    return pl.pallas_call(
        _node_body,
        grid=grid,
        in_specs=[
            pl.BlockSpec((BLKN, D), lambda i: (i, 0)),
            pl.BlockSpec((BLKN, D), lambda i: (i, 0)),
            pl.BlockSpec((BLKN, D), lambda i: (i, 0)),
            pl.BlockSpec((BLKN, 1), lambda i: (i, 0)),
            pl.BlockSpec((BLKN, 1), lambda i: (i, 0)),
            pl.BlockSpec((BLKN, 1), lambda i: (i, 0)),
            full((G, D)), full((G, D)), full((G, D)), full((G, D)),
            full((D, D)), full((1, D)), full((D, D)), full((1, D)),
            full((D, D)), full((D, D)), full((D, D)), full((1, D)),
            full((D, D)), full((1, D)),
        ],
        out_specs=[
            pl.BlockSpec((BLKN, D), lambda i: (i, 0)),
            full((G, D)),
        ],
        out_shape=[
            jax.ShapeDtypeStruct((NP, D), F32),
            jax.ShapeDtypeStruct((G, D), F32),
        ],
        scratch_shapes=[
            pltpu.VMEM((G, D), F32),
            pltpu.VMEM((G, D), F32),
        ],
    )(xn, p0, p1, c0, c1, batch_col, un, u, sue, cne,
      w1b, bn1, wn2, bn2, g1u, g1x, g1e, bg1, wg2, bg2)


def kernel(nodes_in, edge_index, edges_in, global_in, batch_index,
           We1, be1, We2, be2, Wn1, bn1, Wn2, bn2, Wg1, bg1, Wg2, bg2):
    x = jnp.pad(nodes_in, ((0, NP - N), (0, 0)))
    batch_col = jnp.pad(batch_index.astype(jnp.int32), (0, NP - N),
                        constant_values=G)[:, None]
    row = jnp.pad(edge_index[0].astype(jnp.int32), (0, EP - E),
                  constant_values=NP - 1)
    col = jnp.pad(edge_index[1].astype(jnp.int32), (0, EP - E),
                  constant_values=NP - 1)
    e_pad = jnp.pad(edges_in, ((0, EP - E), (0, 0)))
    u = global_in

    xa, xb, xn, ue, un, starts = _preproj(
        x, batch_col, u,
        We1[0:128], We1[128:256], Wn1[0:128], We1[384:512],
        be1[None, :], Wn1[256:384])

    row2d = row.reshape(EP // 128, 128)
    col2d = col.reshape(EP // 128, 128)
    ident = jnp.arange(256, dtype=jnp.int32).reshape(2, 128)
    pre = _sc_gather(xa, xb, row2d, col2d, ident)

    e_new, sue, cne = _edge_mlp(pre, e_pad, row[:, None], starts, ue,
                                We1[256:384], We2, be2[None, :])

    zeros_blk = jnp.zeros((128, D), F32)
    zcol = jnp.zeros((NP // NS, 1), F32)
    onesb = jnp.ones((128, 1), F32)
    aggp, cntp = _sc_scatter(e_new, col2d, zeros_blk, zcol, onesb)

    x_new, u_new = _node_global(
        xn, aggp[:NP], aggp[NP:], cntp[:NP], cntp[NP:], batch_col,
        un, u, sue, cne,
        Wn1[128:256], bn1[None, :], Wn2, bn2[None, :],
        Wg1[0:128], Wg1[128:256], Wg1[256:384], bg1[None, :],
        Wg2, bg2[None, :])

    return (x_new[:N], e_new[:E], u_new)


# speedup vs baseline: 3.8728x; 3.8728x over previous
import functools

import jax
import jax.numpy as jnp
from jax import lax
from jax.experimental import pallas as pl
from jax.experimental.pallas import tpu as pltpu
from jax.experimental.pallas import tpu_sc as plsc

D = 128
N, E, G = 10000, 320000, 8
NP, EP = 10240, 327680
NC, NS = 2, 16
NW = NC * NS
EPW = EP // NW
BLKN = 1024
BLKE = 2048
F32 = jnp.float32


def _preproj_body(x_ref, b_ref, u_ref, wa_ref, wb_ref, wn_ref, wud_ref,
                  be1_ref, wun_ref,
                  xa_ref, xb_ref, xn_ref, ue_ref, un_ref, starts_ref):
    i = pl.program_id(0)
    x = x_ref[...]
    xa_ref[...] = jnp.dot(x, wa_ref[...], preferred_element_type=F32)
    xb_ref[...] = jnp.dot(x, wb_ref[...], preferred_element_type=F32)
    xn_ref[...] = jnp.dot(x, wn_ref[...], preferred_element_type=F32)

    @pl.when(i == 0)
    def _():
        u = u_ref[...]
        ue_ref[...] = jnp.dot(u, wud_ref[...], preferred_element_type=F32) + be1_ref[...]
        un_ref[...] = jnp.dot(u, wun_ref[...], preferred_element_type=F32)
        starts_ref[...] = jnp.zeros_like(starts_ref)

    b = b_ref[...]
    g = lax.broadcasted_iota(jnp.int32, (1, 128), 1)
    cmp = (b < g).astype(jnp.int32)
    s = jnp.sum(cmp, axis=0, keepdims=True)
    starts_ref[...] += jnp.broadcast_to(s, (8, 128))


def _preproj(x, batch_col, u, wa, wb, wn, wud, be1, wun):
    grid = (NP // BLKN,)
    full = lambda shp: pl.BlockSpec(shp, lambda i: (0, 0))
    return pl.pallas_call(
        _preproj_body,
        grid=grid,
        in_specs=[
            pl.BlockSpec((BLKN, D), lambda i: (i, 0)),
            pl.BlockSpec((BLKN, 1), lambda i: (i, 0)),
            full((G, D)), full((D, D)), full((D, D)), full((D, D)),
            full((D, D)), full((1, D)), full((D, D)),
        ],
        out_specs=[
            pl.BlockSpec((BLKN, D), lambda i: (i, 0)),
            pl.BlockSpec((BLKN, D), lambda i: (i, 0)),
            pl.BlockSpec((BLKN, D), lambda i: (i, 0)),
            full((G, D)), full((G, D)), full((8, 128)),
        ],
        out_shape=[
            jax.ShapeDtypeStruct((NP, D), F32),
            jax.ShapeDtypeStruct((NP, D), F32),
            jax.ShapeDtypeStruct((NP, D), F32),
            jax.ShapeDtypeStruct((G, D), F32),
            jax.ShapeDtypeStruct((G, D), F32),
            jax.ShapeDtypeStruct((8, 128), jnp.int32),
        ],
    )(x, batch_col, u, wa, wb, wn, wud, be1, wun)


def _sc_gather_body(xa_hbm, xb_hbm, row2d, col2d, ga_hbm, gb_hbm,
                    idxr, idxc, bufa, bufb, sem):
    s = lax.axis_index("s")
    c = lax.axis_index("c")
    wid = c * NS + s
    base = wid * EPW

    def body(k, carry):
        rb = pl.multiple_of(base // 128 + k * 8, 8)
        pltpu.sync_copy(row2d.at[pl.ds(rb, 8)], idxr)
        pltpu.sync_copy(col2d.at[pl.ds(rb, 8)], idxc)
        for m in range(4):
            cps = []
            for j in range(2):
                cps.append(pltpu.async_copy(
                    xa_hbm.at[idxr.at[2 * m + j]],
                    bufa.at[pl.ds(j * 128, 128)], sem))
                cps.append(pltpu.async_copy(
                    xb_hbm.at[idxc.at[2 * m + j]],
                    bufb.at[pl.ds(j * 128, 128)], sem))
            for cp in cps:
                cp.wait()
            off = pl.multiple_of(base + k * 1024 + m * 256, 256)
            pltpu.sync_copy(bufa, ga_hbm.at[pl.ds(off, 256)])
            pltpu.sync_copy(bufb, gb_hbm.at[pl.ds(off, 256)])
        return carry

    lax.fori_loop(0, EPW // 1024, body, 0)


def _sc_gather(xa, xb, row2d, col2d):
    mesh = plsc.VectorSubcoreMesh(core_axis_name="c", subcore_axis_name="s")
    f = pl.kernel(
        _sc_gather_body,
        out_type=[
            jax.ShapeDtypeStruct((EP, D), F32),
            jax.ShapeDtypeStruct((EP, D), F32),
        ],
        mesh=mesh,
        scratch_types=[
            pltpu.VMEM((8, 128), jnp.int32),
            pltpu.VMEM((8, 128), jnp.int32),
            pltpu.VMEM((256, D), F32),
            pltpu.VMEM((256, D), F32),
            pltpu.SemaphoreType.DMA,
        ],
    )
    return f(xa, xb, row2d, col2d)


def _edge_body(ga_ref, gb_ref, e_ref, r_ref, s_ref, ue_ref, w1c_ref, w2_ref,
               be2_ref, out_ref, sue_ref, cne_ref):
    i = pl.program_id(0)
    r = r_ref[...]
    gi = i * BLKE + lax.broadcasted_iota(jnp.int32, (BLKE, 1), 0)
    valid = (gi < E).astype(F32)
    bi = jnp.zeros((BLKE, 1), jnp.int32)
    for gg in range(1, 8):
        bi = bi + (r >= s_ref[0:1, gg:gg + 1]).astype(jnp.int32)
    onehot = (bi == lax.broadcasted_iota(jnp.int32, (1, 8), 1)).astype(F32)
    onehot = onehot * valid
    uterm = jnp.dot(onehot, ue_ref[...], preferred_element_type=F32)
    h = jnp.maximum(ga_ref[...] + gb_ref[...] +
                    jnp.dot(e_ref[...], w1c_ref[...], preferred_element_type=F32) +
                    uterm, 0.0)
    e_new = jnp.dot(h, w2_ref[...], preferred_element_type=F32) + be2_ref[...]
    out_ref[...] = e_new

    @pl.when(i == 0)
    def _():
        sue_ref[...] = jnp.zeros_like(sue_ref)
        cne_ref[...] = jnp.zeros_like(cne_ref)
    sue_ref[...] += lax.dot_general(onehot, e_new, (((0,), (0,)), ((), ())),
                                    preferred_element_type=F32)
    ones = jnp.ones((BLKE, 128), F32)
    cne_ref[...] += lax.dot_general(onehot, ones, (((0,), (0,)), ((), ())),
                                    preferred_element_type=F32)


def _edge_mlp(ga, gb, e_pad, row_col, starts, ue, w1c, w2, be2):
    grid = (EP // BLKE,)
    full = lambda shp: pl.BlockSpec(shp, lambda i: (0, 0))
    return pl.pallas_call(
        _edge_body,
        grid=grid,
        in_specs=[
            pl.BlockSpec((BLKE, D), lambda i: (i, 0)),
            pl.BlockSpec((BLKE, D), lambda i: (i, 0)),
            pl.BlockSpec((BLKE, D), lambda i: (i, 0)),
            pl.BlockSpec((BLKE, 1), lambda i: (i, 0)),
            full((8, 128)), full((G, D)), full((D, D)), full((D, D)),
            full((1, D)),
        ],
        out_specs=[
            pl.BlockSpec((BLKE, D), lambda i: (i, 0)),
            full((G, D)), full((G, D)),
        ],
        out_shape=[
            jax.ShapeDtypeStruct((EP, D), F32),
            jax.ShapeDtypeStruct((G, D), F32),
            jax.ShapeDtypeStruct((G, D), F32),
        ],
    )(ga, gb, e_pad, row_col, starts, ue, w1c, w2, be2)


def _sc_scatter_body(enew, col2d, zeros_hbm, zcol_hbm, ones_hbm,
                     aggp, cntp,
                     buf, idxc, cbuf, onesv, aggS, cntS):
    s = lax.axis_index("s")
    c = lax.axis_index("c")
    wid = c * NS + s
    rows = NP // NS
    pltpu.sync_copy(ones_hbm, onesv)

    pltpu.sync_copy(zeros_hbm, buf)
    for t in range(rows // 128):
        pltpu.sync_copy(
            buf, aggS.at[pl.ds(pl.multiple_of(s * rows + t * 128, 128), 128)])
    pltpu.sync_copy(zcol_hbm, cbuf)
    pltpu.sync_copy(cbuf, cntS.at[pl.ds(pl.multiple_of(s * rows, rows), rows)])
    plsc.subcore_barrier()

    base128 = wid * (EPW // 128)

    def body(k, carry):
        blk = pl.multiple_of(base128 + k * 8, 8)
        pltpu.sync_copy(col2d.at[pl.ds(blk, 8)], idxc)
        for m in range(8):
            pltpu.sync_copy(
                enew.at[pl.ds(pl.multiple_of((blk + m) * 128, 128), 128)], buf)
            pltpu.sync_copy(buf, aggS.at[idxc.at[m]], add=True)
            pltpu.sync_copy(onesv, cntS.at[idxc.at[m]], add=True)
        return carry

    lax.fori_loop(0, EPW // 1024, body, 0)
    plsc.subcore_barrier()

    for t in range(rows // 128):
        r = pl.multiple_of(s * rows + t * 128, 128)
        pltpu.sync_copy(aggS.at[pl.ds(r, 128)], buf)
        pltpu.sync_copy(buf, aggp.at[pl.ds(pl.multiple_of(c * NP + r, 128), 128)])
    pltpu.sync_copy(cntS.at[pl.ds(pl.multiple_of(s * rows, rows), rows)], cbuf)
    pltpu.sync_copy(
        cbuf, cntp.at[pl.ds(pl.multiple_of(c * NP + s * rows, rows), rows)])


def _sc_scatter(e_new, col2d, zeros_blk, zcol, onesb):
    mesh = plsc.VectorSubcoreMesh(core_axis_name="c", subcore_axis_name="s")
    f = pl.kernel(
        _sc_scatter_body,
        out_type=[
            jax.ShapeDtypeStruct((NC * NP, D), F32),
            jax.ShapeDtypeStruct((NC * NP,), F32),
        ],
        mesh=mesh,
        scratch_types=[
            pltpu.VMEM((128, D), F32),
            pltpu.VMEM((8, 128), jnp.int32),
            pltpu.VMEM((NP // NS,), F32),
            pltpu.VMEM((128,), F32),
            pltpu.VMEM_SHARED((NP, D), F32),
            pltpu.VMEM_SHARED((NP,), F32),
        ],
    )
    return f(e_new, col2d, zeros_blk, zcol, onesb)


def _node_body(xn_ref, p0_ref, p1_ref, c0_ref, c1_ref, b_ref, un_ref,
               u_ref, sue_ref, cne_ref, w1b_ref, bn1_ref, wn2_ref, bn2_ref,
               g1u_ref, g1x_ref, g1e_ref, bg1_ref, wg2_ref, bg2_ref,
               out_ref, uout_ref, sx_ref, cn_ref):
    i = pl.program_id(0)

    @pl.when(i == 0)
    def _():
        sx_ref[...] = jnp.zeros_like(sx_ref)
        cn_ref[...] = jnp.zeros_like(cn_ref)

    cnt = jnp.maximum(c0_ref[...] + c1_ref[...], 1.0)
    agg = (p0_ref[...] + p1_ref[...]) / cnt
    b = b_ref[...]
    onehot = (b == lax.broadcasted_iota(jnp.int32, (1, 8), 1)).astype(F32)
    z = (xn_ref[...] +
         jnp.dot(agg, w1b_ref[...], preferred_element_type=F32) +
         jnp.dot(onehot, un_ref[...], preferred_element_type=F32) +
         bn1_ref[...])
    h = jnp.maximum(z, 0.0)
    x_new = jnp.dot(h, wn2_ref[...], preferred_element_type=F32) + bn2_ref[...]
    out_ref[...] = x_new
    sx_ref[...] += lax.dot_general(onehot, x_new, (((0,), (0,)), ((), ())),
                                   preferred_element_type=F32)
    ones = jnp.ones((BLKN, 128), F32)
    cn_ref[...] += lax.dot_general(onehot, ones, (((0,), (0,)), ((), ())),
                                   preferred_element_type=F32)

    @pl.when(i == pl.num_programs(0) - 1)
    def _():
        sxm = sx_ref[...] / jnp.maximum(cn_ref[...], 1.0)
        sem = sue_ref[...] / jnp.maximum(cne_ref[...], 1.0)
        zg = (jnp.dot(u_ref[...], g1u_ref[...], preferred_element_type=F32) +
              jnp.dot(sxm, g1x_ref[...], preferred_element_type=F32) +
              jnp.dot(sem, g1e_ref[...], preferred_element_type=F32) +
              bg1_ref[...])
        hg = jnp.maximum(zg, 0.0)
        uout_ref[...] = jnp.dot(hg, wg2_ref[...], preferred_element_type=F32) + bg2_ref[...]


def _node_global(xn, p0, p1, c0, c1, batch_col, un, u, sue, cne,
                 w1b, bn1, wn2, bn2, g1u, g1x, g1e, bg1, wg2, bg2):
    grid = (NP // BLKN,)
    full = lambda shp: pl.BlockSpec(shp, lambda i: (0, 0))
    return pl.pallas_call(
        _node_body,
        grid=grid,
        in_specs=[
            pl.BlockSpec((BLKN, D), lambda i: (i, 0)),
            pl.BlockSpec((BLKN, D), lambda i: (i, 0)),
            pl.BlockSpec((BLKN, D), lambda i: (i, 0)),
            pl.BlockSpec((BLKN, 1), lambda i: (i, 0)),
            pl.BlockSpec((BLKN, 1), lambda i: (i, 0)),
            pl.BlockSpec((BLKN, 1), lambda i: (i, 0)),
            full((G, D)), full((G, D)), full((G, D)), full((G, D)),
            full((D, D)), full((1, D)), full((D, D)), full((1, D)),
            full((D, D)), full((D, D)), full((D, D)), full((1, D)),
            full((D, D)), full((1, D)),
        ],
        out_specs=[
            pl.BlockSpec((BLKN, D), lambda i: (i, 0)),
            full((G, D)),
        ],
        out_shape=[
            jax.ShapeDtypeStruct((NP, D), F32),
            jax.ShapeDtypeStruct((G, D), F32),
        ],
        scratch_shapes=[
            pltpu.VMEM((G, D), F32),
            pltpu.VMEM((G, D), F32),
        ],
    )(xn, p0, p1, c0, c1, batch_col, un, u, sue, cne,
      w1b, bn1, wn2, bn2, g1u, g1x, g1e, bg1, wg2, bg2)


def kernel(nodes_in, edge_index, edges_in, global_in, batch_index,
           We1, be1, We2, be2, Wn1, bn1, Wn2, bn2, Wg1, bg1, Wg2, bg2):
    x = jnp.pad(nodes_in, ((0, NP - N), (0, 0)))
    batch_col = jnp.pad(batch_index.astype(jnp.int32), (0, NP - N),
                        constant_values=G)[:, None]
    row = jnp.pad(edge_index[0].astype(jnp.int32), (0, EP - E),
                  constant_values=NP - 1)
    col = jnp.pad(edge_index[1].astype(jnp.int32), (0, EP - E),
                  constant_values=NP - 1)
    e_pad = jnp.pad(edges_in, ((0, EP - E), (0, 0)))
    u = global_in

    xa, xb, xn, ue, un, starts = _preproj(
        x, batch_col, u,
        We1[0:128], We1[128:256], Wn1[0:128], We1[384:512],
        be1[None, :], Wn1[256:384])

    row2d = row.reshape(EP // 128, 128)
    col2d = col.reshape(EP // 128, 128)
    ga, gb = _sc_gather(xa, xb, row2d, col2d)

    e_new, sue, cne = _edge_mlp(ga, gb, e_pad, row[:, None], starts, ue,
                                We1[256:384], We2, be2[None, :])

    zeros_blk = jnp.zeros((128, D), F32)
    zcol = jnp.zeros((NP // NS,), F32)
    onesb = jnp.ones((128,), F32)
    aggp, cntp = _sc_scatter(e_new, col2d, zeros_blk, zcol, onesb)
    cntp = cntp[:, None]

    x_new, u_new = _node_global(
        xn, aggp[:NP], aggp[NP:], cntp[:NP], cntp[NP:], batch_col,
        un, u, sue, cne,
        Wn1[128:256], bn1[None, :], Wn2, bn2[None, :],
        Wg1[0:128], Wg1[128:256], Wg1[256:384], bg1[None, :],
        Wg2, bg2[None, :])

    return (x_new[:N], e_new[:E], u_new)


# speedup vs baseline: 4.3670x; 1.1276x over previous
import functools

import jax
import jax.numpy as jnp
from jax import lax
from jax.experimental import pallas as pl
from jax.experimental.pallas import tpu as pltpu
from jax.experimental.pallas import tpu_sc as plsc

D = 128
N, E, G = 10000, 320000, 8
NP, EP = 10240, 327680
NC, NS = 2, 16
NW = NC * NS
EPW = EP // NW
BLKN = 1024
BLKE = 2048
F32 = jnp.float32


def _preproj_body(x_ref, b_ref, u_ref, wa_ref, wb_ref, wn_ref, wud_ref,
                  be1_ref, wun_ref,
                  xa_ref, xb_ref, xn_ref, ue_ref, un_ref, starts_ref):
    i = pl.program_id(0)
    x = x_ref[...]
    xa_ref[...] = jnp.dot(x, wa_ref[...], preferred_element_type=F32)
    xb_ref[...] = jnp.dot(x, wb_ref[...], preferred_element_type=F32)
    xn_ref[...] = jnp.dot(x, wn_ref[...], preferred_element_type=F32)

    @pl.when(i == 0)
    def _():
        u = u_ref[...]
        ue_ref[...] = jnp.dot(u, wud_ref[...], preferred_element_type=F32) + be1_ref[...]
        un_ref[...] = jnp.dot(u, wun_ref[...], preferred_element_type=F32)
        starts_ref[...] = jnp.zeros_like(starts_ref)

    b = b_ref[...]
    g = lax.broadcasted_iota(jnp.int32, (1, 128), 1)
    cmp = (b < g).astype(jnp.int32)
    s = jnp.sum(cmp, axis=0, keepdims=True)
    starts_ref[...] += jnp.broadcast_to(s, (8, 128))


def _preproj(x, batch_col, u, wa, wb, wn, wud, be1, wun):
    grid = (NP // BLKN,)
    full = lambda shp: pl.BlockSpec(shp, lambda i: (0, 0))
    return pl.pallas_call(
        _preproj_body,
        grid=grid,
        in_specs=[
            pl.BlockSpec((BLKN, D), lambda i: (i, 0)),
            pl.BlockSpec((BLKN, 1), lambda i: (i, 0)),
            full((G, D)), full((D, D)), full((D, D)), full((D, D)),
            full((D, D)), full((1, D)), full((D, D)),
        ],
        out_specs=[
            pl.BlockSpec((BLKN, D), lambda i: (i, 0)),
            pl.BlockSpec((BLKN, D), lambda i: (i, 0)),
            pl.BlockSpec((BLKN, D), lambda i: (i, 0)),
            full((G, D)), full((G, D)), full((8, 128)),
        ],
        out_shape=[
            jax.ShapeDtypeStruct((NP, D), F32),
            jax.ShapeDtypeStruct((NP, D), F32),
            jax.ShapeDtypeStruct((NP, D), F32),
            jax.ShapeDtypeStruct((G, D), F32),
            jax.ShapeDtypeStruct((G, D), F32),
            jax.ShapeDtypeStruct((8, 128), jnp.int32),
        ],
    )(x, batch_col, u, wa, wb, wn, wud, be1, wun)


def _sc_gather_body(xa_hbm, xb_hbm, row2d, col2d, ga_hbm, gb_hbm,
                    idxr, idxc, bufa, bufb, sem, ssem):
    s = lax.axis_index("s")
    c = lax.axis_index("c")
    wid = c * NS + s
    base = wid * EPW

    NB = 3

    def body(k, carry):
        rb = pl.multiple_of(base // 128 + k * 8, 8)
        i0 = pltpu.async_copy(row2d.at[pl.ds(rb, 8)], idxr, sem)
        i1 = pltpu.async_copy(col2d.at[pl.ds(rb, 8)], idxc, sem)
        i0.wait()
        i1.wait()
        gcps = [None] * 8
        scps = [None] * 8

        def start_store(p):
            psl = p % NB
            gcps[p][0].wait()
            gcps[p][1].wait()
            off = pl.multiple_of(base + k * 1024 + p * 128, 128)
            scps[p] = (
                pltpu.async_copy(bufa.at[psl], ga_hbm.at[pl.ds(off, 128)], ssem),
                pltpu.async_copy(bufb.at[psl], gb_hbm.at[pl.ds(off, 128)], ssem),
            )

        for m in range(8):
            sl = m % NB
            if m >= NB:
                scps[m - NB][0].wait()
                scps[m - NB][1].wait()
            gcps[m] = (
                pltpu.async_copy(xa_hbm.at[idxr.at[m]], bufa.at[sl], sem),
                pltpu.async_copy(xb_hbm.at[idxc.at[m]], bufb.at[sl], sem),
            )
            if m >= 1:
                start_store(m - 1)
        start_store(7)
        for p in range(8 - NB, 8):
            scps[p][0].wait()
            scps[p][1].wait()
        return carry

    lax.fori_loop(0, EPW // 1024, body, 0)


def _sc_gather(xa, xb, row2d, col2d):
    mesh = plsc.VectorSubcoreMesh(core_axis_name="c", subcore_axis_name="s")
    f = pl.kernel(
        _sc_gather_body,
        out_type=[
            jax.ShapeDtypeStruct((EP, D), F32),
            jax.ShapeDtypeStruct((EP, D), F32),
        ],
        mesh=mesh,
        scratch_types=[
            pltpu.VMEM((8, 128), jnp.int32),
            pltpu.VMEM((8, 128), jnp.int32),
            pltpu.VMEM((3, 128, D), F32),
            pltpu.VMEM((3, 128, D), F32),
            pltpu.SemaphoreType.DMA,
            pltpu.SemaphoreType.DMA,
        ],
    )
    return f(xa, xb, row2d, col2d)


def _edge_body(ga_ref, gb_ref, e_ref, r_ref, s_ref, ue_ref, w1c_ref, w2_ref,
               be2_ref, out_ref, sue_ref, cne_ref):
    i = pl.program_id(0)
    r = r_ref[...]
    gi = i * BLKE + lax.broadcasted_iota(jnp.int32, (BLKE, 1), 0)
    valid = (gi < E).astype(F32)
    bi = jnp.zeros((BLKE, 1), jnp.int32)
    for gg in range(1, 8):
        bi = bi + (r >= s_ref[0:1, gg:gg + 1]).astype(jnp.int32)
    onehot = (bi == lax.broadcasted_iota(jnp.int32, (1, 8), 1)).astype(F32)
    onehot = onehot * valid
    uterm = jnp.dot(onehot, ue_ref[...], preferred_element_type=F32)
    h = jnp.maximum(ga_ref[...] + gb_ref[...] +
                    jnp.dot(e_ref[...], w1c_ref[...], preferred_element_type=F32) +
                    uterm, 0.0)
    e_new = jnp.dot(h, w2_ref[...], preferred_element_type=F32) + be2_ref[...]
    out_ref[...] = e_new

    @pl.when(i == 0)
    def _():
        sue_ref[...] = jnp.zeros_like(sue_ref)
        cne_ref[...] = jnp.zeros_like(cne_ref)
    sue_ref[...] += lax.dot_general(onehot, e_new, (((0,), (0,)), ((), ())),
                                    preferred_element_type=F32)
    ones = jnp.ones((BLKE, 128), F32)
    cne_ref[...] += lax.dot_general(onehot, ones, (((0,), (0,)), ((), ())),
                                    preferred_element_type=F32)


def _edge_mlp(ga, gb, e_pad, row_col, starts, ue, w1c, w2, be2):
    grid = (EP // BLKE,)
    full = lambda shp: pl.BlockSpec(shp, lambda i: (0, 0))
    return pl.pallas_call(
        _edge_body,
        grid=grid,
        in_specs=[
            pl.BlockSpec((BLKE, D), lambda i: (i, 0)),
            pl.BlockSpec((BLKE, D), lambda i: (i, 0)),
            pl.BlockSpec((BLKE, D), lambda i: (i, 0)),
            pl.BlockSpec((BLKE, 1), lambda i: (i, 0)),
            full((8, 128)), full((G, D)), full((D, D)), full((D, D)),
            full((1, D)),
        ],
        out_specs=[
            pl.BlockSpec((BLKE, D), lambda i: (i, 0)),
            full((G, D)), full((G, D)),
        ],
        out_shape=[
            jax.ShapeDtypeStruct((EP, D), F32),
            jax.ShapeDtypeStruct((G, D), F32),
            jax.ShapeDtypeStruct((G, D), F32),
        ],
    )(ga, gb, e_pad, row_col, starts, ue, w1c, w2, be2)


def _sc_scatter_body(enew, col2d, zeros_hbm, zcol_hbm, ones_hbm,
                     aggp, cntp,
                     buf, idxc, cbuf, onesv, aggS, cntS, lsem, asem):
    s = lax.axis_index("s")
    c = lax.axis_index("c")
    wid = c * NS + s
    rows = NP // NS
    pltpu.sync_copy(ones_hbm, onesv)

    pltpu.sync_copy(zeros_hbm, buf.at[0])
    for t in range(rows // 128):
        pltpu.sync_copy(
            buf.at[0],
            aggS.at[pl.ds(pl.multiple_of(s * rows + t * 128, 128), 128)])
    pltpu.sync_copy(zcol_hbm, cbuf)
    pltpu.sync_copy(cbuf, cntS.at[pl.ds(pl.multiple_of(s * rows, rows), rows)])
    plsc.subcore_barrier()

    base128 = wid * (EPW // 128)
    NB = 2

    def body(k, carry):
        blk = pl.multiple_of(base128 + k * 8, 8)
        pltpu.sync_copy(col2d.at[pl.ds(blk, 8)], idxc)
        lcps = [None] * 8
        acps = [None] * 8
        ccps = [None] * 8

        def start_add(p):
            psl = p % NB
            lcps[p].wait()
            acps[p] = pltpu.async_copy(buf.at[psl], aggS.at[idxc.at[p]],
                                       asem, add=True)
            ccps[p] = pltpu.async_copy(onesv, cntS.at[idxc.at[p]],
                                       asem, add=True)

        for m in range(8):
            sl = m % NB
            if m >= NB:
                acps[m - NB].wait()
                ccps[m - NB].wait()
            off = pl.multiple_of((blk + m) * 128, 128)
            lcps[m] = pltpu.async_copy(enew.at[pl.ds(off, 128)], buf.at[sl],
                                       lsem)
            if m >= 1:
                start_add(m - 1)
        start_add(7)
        for p in range(8 - NB, 8):
            acps[p].wait()
            ccps[p].wait()
        return carry

    lax.fori_loop(0, EPW // 1024, body, 0)
    plsc.subcore_barrier()

    for t in range(rows // 128):
        r = pl.multiple_of(s * rows + t * 128, 128)
        pltpu.sync_copy(aggS.at[pl.ds(r, 128)], buf.at[0])
        pltpu.sync_copy(buf.at[0],
                        aggp.at[pl.ds(pl.multiple_of(c * NP + r, 128), 128)])
    pltpu.sync_copy(cntS.at[pl.ds(pl.multiple_of(s * rows, rows), rows)], cbuf)
    pltpu.sync_copy(
        cbuf, cntp.at[pl.ds(pl.multiple_of(c * NP + s * rows, rows), rows)])


def _sc_scatter(e_new, col2d, zeros_blk, zcol, onesb):
    mesh = plsc.VectorSubcoreMesh(core_axis_name="c", subcore_axis_name="s")
    f = pl.kernel(
        _sc_scatter_body,
        out_type=[
            jax.ShapeDtypeStruct((NC * NP, D), F32),
            jax.ShapeDtypeStruct((NC * NP,), F32),
        ],
        mesh=mesh,
        scratch_types=[
            pltpu.VMEM((2, 128, D), F32),
            pltpu.VMEM((8, 128), jnp.int32),
            pltpu.VMEM((NP // NS,), F32),
            pltpu.VMEM((128,), F32),
            pltpu.VMEM_SHARED((NP, D), F32),
            pltpu.VMEM_SHARED((NP,), F32),
            pltpu.SemaphoreType.DMA,
            pltpu.SemaphoreType.DMA,
        ],
    )
    return f(e_new, col2d, zeros_blk, zcol, onesb)


def _node_body(xn_ref, p0_ref, p1_ref, c0_ref, c1_ref, b_ref, un_ref,
               u_ref, sue_ref, cne_ref, w1b_ref, bn1_ref, wn2_ref, bn2_ref,
               g1u_ref, g1x_ref, g1e_ref, bg1_ref, wg2_ref, bg2_ref,
               out_ref, uout_ref, sx_ref, cn_ref):
    i = pl.program_id(0)

    @pl.when(i == 0)
    def _():
        sx_ref[...] = jnp.zeros_like(sx_ref)
        cn_ref[...] = jnp.zeros_like(cn_ref)

    cnt = jnp.maximum(c0_ref[...] + c1_ref[...], 1.0)
    agg = (p0_ref[...] + p1_ref[...]) / cnt
    b = b_ref[...]
    onehot = (b == lax.broadcasted_iota(jnp.int32, (1, 8), 1)).astype(F32)
    z = (xn_ref[...] +
         jnp.dot(agg, w1b_ref[...], preferred_element_type=F32) +
         jnp.dot(onehot, un_ref[...], preferred_element_type=F32) +
         bn1_ref[...])
    h = jnp.maximum(z, 0.0)
    x_new = jnp.dot(h, wn2_ref[...], preferred_element_type=F32) + bn2_ref[...]
    out_ref[...] = x_new
    sx_ref[...] += lax.dot_general(onehot, x_new, (((0,), (0,)), ((), ())),
                                   preferred_element_type=F32)
    ones = jnp.ones((BLKN, 128), F32)
    cn_ref[...] += lax.dot_general(onehot, ones, (((0,), (0,)), ((), ())),
                                   preferred_element_type=F32)

    @pl.when(i == pl.num_programs(0) - 1)
    def _():
        sxm = sx_ref[...] / jnp.maximum(cn_ref[...], 1.0)
        sem = sue_ref[...] / jnp.maximum(cne_ref[...], 1.0)
        zg = (jnp.dot(u_ref[...], g1u_ref[...], preferred_element_type=F32) +
              jnp.dot(sxm, g1x_ref[...], preferred_element_type=F32) +
              jnp.dot(sem, g1e_ref[...], preferred_element_type=F32) +
              bg1_ref[...])
        hg = jnp.maximum(zg, 0.0)
        uout_ref[...] = jnp.dot(hg, wg2_ref[...], preferred_element_type=F32) + bg2_ref[...]


def _node_global(xn, p0, p1, c0, c1, batch_col, un, u, sue, cne,
                 w1b, bn1, wn2, bn2, g1u, g1x, g1e, bg1, wg2, bg2):
    grid = (NP // BLKN,)
    full = lambda shp: pl.BlockSpec(shp, lambda i: (0, 0))
    return pl.pallas_call(
        _node_body,
        grid=grid,
        in_specs=[
            pl.BlockSpec((BLKN, D), lambda i: (i, 0)),
            pl.BlockSpec((BLKN, D), lambda i: (i, 0)),
            pl.BlockSpec((BLKN, D), lambda i: (i, 0)),
            pl.BlockSpec((BLKN, 1), lambda i: (i, 0)),
            pl.BlockSpec((BLKN, 1), lambda i: (i, 0)),
            pl.BlockSpec((BLKN, 1), lambda i: (i, 0)),
            full((G, D)), full((G, D)), full((G, D)), full((G, D)),
            full((D, D)), full((1, D)), full((D, D)), full((1, D)),
            full((D, D)), full((D, D)), full((D, D)), full((1, D)),
            full((D, D)), full((1, D)),
        ],
        out_specs=[
            pl.BlockSpec((BLKN, D), lambda i: (i, 0)),
            full((G, D)),
        ],
        out_shape=[
            jax.ShapeDtypeStruct((NP, D), F32),
            jax.ShapeDtypeStruct((G, D), F32),
        ],
        scratch_shapes=[
            pltpu.VMEM((G, D), F32),
            pltpu.VMEM((G, D), F32),
        ],
    )(xn, p0, p1, c0, c1, batch_col, un, u, sue, cne,
      w1b, bn1, wn2, bn2, g1u, g1x, g1e, bg1, wg2, bg2)


def kernel(nodes_in, edge_index, edges_in, global_in, batch_index,
           We1, be1, We2, be2, Wn1, bn1, Wn2, bn2, Wg1, bg1, Wg2, bg2):
    x = jnp.pad(nodes_in, ((0, NP - N), (0, 0)))
    batch_col = jnp.pad(batch_index.astype(jnp.int32), (0, NP - N),
                        constant_values=G)[:, None]
    row = jnp.pad(edge_index[0].astype(jnp.int32), (0, EP - E),
                  constant_values=NP - 1)
    col = jnp.pad(edge_index[1].astype(jnp.int32), (0, EP - E),
                  constant_values=NP - 1)
    e_pad = jnp.pad(edges_in, ((0, EP - E), (0, 0)))
    u = global_in

    xa, xb, xn, ue, un, starts = _preproj(
        x, batch_col, u,
        We1[0:128], We1[128:256], Wn1[0:128], We1[384:512],
        be1[None, :], Wn1[256:384])

    row2d = row.reshape(EP // 128, 128)
    col2d = col.reshape(EP // 128, 128)
    ga, gb = _sc_gather(xa, xb, row2d, col2d)

    e_new, sue, cne = _edge_mlp(ga, gb, e_pad, row[:, None], starts, ue,
                                We1[256:384], We2, be2[None, :])

    zeros_blk = jnp.zeros((128, D), F32)
    zcol = jnp.zeros((NP // NS,), F32)
    onesb = jnp.ones((128,), F32)
    aggp, cntp = _sc_scatter(e_new, col2d, zeros_blk, zcol, onesb)
    cntp = cntp[:, None]

    x_new, u_new = _node_global(
        xn, aggp[:NP], aggp[NP:], cntp[:NP], cntp[NP:], batch_col,
        un, u, sue, cne,
        Wn1[128:256], bn1[None, :], Wn2, bn2[None, :],
        Wg1[0:128], Wg1[128:256], Wg1[256:384], bg1[None, :],
        Wg2, bg2[None, :])

    return (x_new[:N], e_new[:E], u_new)


# speedup vs baseline: 4.4352x; 1.0156x over previous
import functools

import jax
import jax.numpy as jnp
from jax import lax
from jax.experimental import pallas as pl
from jax.experimental.pallas import tpu as pltpu
from jax.experimental.pallas import tpu_sc as plsc

D = 128
N, E, G = 10000, 320000, 8
NP, EP = 10240, 327680
NC, NS = 2, 16
NW = NC * NS
EPW = EP // NW
BLKN = 1024
BLKE = 2048
F32 = jnp.float32


def _preproj_body(x_ref, b_ref, u_ref, wa_ref, wb_ref, wn_ref, wud_ref,
                  be1_ref, wun_ref,
                  xa_ref, xb_ref, xn_ref, ue_ref, un_ref, starts_ref):
    i = pl.program_id(0)
    x = x_ref[...]
    xa_ref[...] = jnp.dot(x, wa_ref[...], preferred_element_type=F32)
    xb_ref[...] = jnp.dot(x, wb_ref[...], preferred_element_type=F32)
    xn_ref[...] = jnp.dot(x, wn_ref[...], preferred_element_type=F32)

    @pl.when(i == 0)
    def _():
        u = u_ref[...]
        ue_ref[...] = jnp.dot(u, wud_ref[...], preferred_element_type=F32) + be1_ref[...]
        un_ref[...] = jnp.dot(u, wun_ref[...], preferred_element_type=F32)
        starts_ref[...] = jnp.zeros_like(starts_ref)

    b = b_ref[...]
    g = lax.broadcasted_iota(jnp.int32, (1, 128), 1)
    cmp = (b < g).astype(jnp.int32)
    s = jnp.sum(cmp, axis=0, keepdims=True)
    starts_ref[...] += jnp.broadcast_to(s, (8, 128))


def _preproj(x, batch_col, u, wa, wb, wn, wud, be1, wun):
    grid = (NP // BLKN,)
    full = lambda shp: pl.BlockSpec(shp, lambda i: (0, 0))
    return pl.pallas_call(
        _preproj_body,
        grid=grid,
        in_specs=[
            pl.BlockSpec((BLKN, D), lambda i: (i, 0)),
            pl.BlockSpec((BLKN, 1), lambda i: (i, 0)),
            full((G, D)), full((D, D)), full((D, D)), full((D, D)),
            full((D, D)), full((1, D)), full((D, D)),
        ],
        out_specs=[
            pl.BlockSpec((BLKN, D), lambda i: (i, 0)),
            pl.BlockSpec((BLKN, D), lambda i: (i, 0)),
            pl.BlockSpec((BLKN, D), lambda i: (i, 0)),
            full((G, D)), full((G, D)), full((8, 128)),
        ],
        out_shape=[
            jax.ShapeDtypeStruct((NP, D), F32),
            jax.ShapeDtypeStruct((NP, D), F32),
            jax.ShapeDtypeStruct((NP, D), F32),
            jax.ShapeDtypeStruct((G, D), F32),
            jax.ShapeDtypeStruct((G, D), F32),
            jax.ShapeDtypeStruct((8, 128), jnp.int32),
        ],
    )(x, batch_col, u, wa, wb, wn, wud, be1, wun)


def _sc_gather_body(xa_hbm, xb_hbm, row2d, col2d, ga_hbm, gb_hbm,
                    idxr, idxc, bufa, bufb, sem, ssem):
    s = lax.axis_index("s")
    c = lax.axis_index("c")
    wid = c * NS + s
    base = wid * EPW

    NB = 3
    toff = jnp.zeros((16,), jnp.int32) + c * NP

    def body(k, carry):
        rb = pl.multiple_of(base // 128 + k * 8, 8)
        i0 = pltpu.async_copy(row2d.at[pl.ds(rb, 8)], idxr, sem)
        i1 = pltpu.async_copy(col2d.at[pl.ds(rb, 8)], idxc, sem)
        i0.wait()
        i1.wait()
        for r in range(8):
            for j in range(8):
                sl16 = pl.ds(j * 16, 16)
                idxr[r, sl16] = idxr[r, sl16] + toff
                idxc[r, sl16] = idxc[r, sl16] + toff
        gcps = [None] * 8
        scps = [None] * 8

        def start_store(p):
            psl = p % NB
            gcps[p][0].wait()
            gcps[p][1].wait()
            off = pl.multiple_of(base + k * 1024 + p * 128, 128)
            scps[p] = (
                pltpu.async_copy(bufa.at[psl], ga_hbm.at[pl.ds(off, 128)], ssem),
                pltpu.async_copy(bufb.at[psl], gb_hbm.at[pl.ds(off, 128)], ssem),
            )

        for m in range(8):
            sl = m % NB
            if m >= NB:
                scps[m - NB][0].wait()
                scps[m - NB][1].wait()
            gcps[m] = (
                pltpu.async_copy(xa_hbm.at[idxr.at[m]], bufa.at[sl], sem),
                pltpu.async_copy(xb_hbm.at[idxc.at[m]], bufb.at[sl], sem),
            )
            if m >= 1:
                start_store(m - 1)
        start_store(7)
        for p in range(8 - NB, 8):
            scps[p][0].wait()
            scps[p][1].wait()
        return carry

    lax.fori_loop(0, EPW // 1024, body, 0)


def _sc_gather(xa, xb, row2d, col2d):
    mesh = plsc.VectorSubcoreMesh(core_axis_name="c", subcore_axis_name="s")
    f = pl.kernel(
        _sc_gather_body,
        out_type=[
            jax.ShapeDtypeStruct((EP, D), F32),
            jax.ShapeDtypeStruct((EP, D), F32),
        ],
        mesh=mesh,
        scratch_types=[
            pltpu.VMEM((8, 128), jnp.int32),
            pltpu.VMEM((8, 128), jnp.int32),
            pltpu.VMEM((3, 128, D), F32),
            pltpu.VMEM((3, 128, D), F32),
            pltpu.SemaphoreType.DMA,
            pltpu.SemaphoreType.DMA,
        ],
    )
    return f(jnp.concatenate([xa, xa], axis=0),
             jnp.concatenate([xb, xb], axis=0), row2d, col2d)


def _edge_body(ga_ref, gb_ref, e_ref, r_ref, s_ref, ue_ref, w1c_ref, w2_ref,
               be2_ref, out_ref, sue_ref, cne_ref):
    i = pl.program_id(0)
    r = r_ref[...]
    gi = i * BLKE + lax.broadcasted_iota(jnp.int32, (BLKE, 1), 0)
    valid = (gi < E).astype(F32)
    bi = jnp.zeros((BLKE, 1), jnp.int32)
    for gg in range(1, 8):
        bi = bi + (r >= s_ref[0:1, gg:gg + 1]).astype(jnp.int32)
    onehot = (bi == lax.broadcasted_iota(jnp.int32, (1, 8), 1)).astype(F32)
    onehot = onehot * valid
    uterm = jnp.dot(onehot, ue_ref[...], preferred_element_type=F32)
    h = jnp.maximum(ga_ref[...] + gb_ref[...] +
                    jnp.dot(e_ref[...], w1c_ref[...], preferred_element_type=F32) +
                    uterm, 0.0)
    e_new = jnp.dot(h, w2_ref[...], preferred_element_type=F32) + be2_ref[...]
    out_ref[...] = e_new

    @pl.when(i == 0)
    def _():
        sue_ref[...] = jnp.zeros_like(sue_ref)
        cne_ref[...] = jnp.zeros_like(cne_ref)
    sue_ref[...] += lax.dot_general(onehot, e_new, (((0,), (0,)), ((), ())),
                                    preferred_element_type=F32)
    ones = jnp.ones((BLKE, 128), F32)
    cne_ref[...] += lax.dot_general(onehot, ones, (((0,), (0,)), ((), ())),
                                    preferred_element_type=F32)


def _edge_mlp(ga, gb, e_pad, row_col, starts, ue, w1c, w2, be2):
    grid = (EP // BLKE,)
    full = lambda shp: pl.BlockSpec(shp, lambda i: (0, 0))
    return pl.pallas_call(
        _edge_body,
        grid=grid,
        in_specs=[
            pl.BlockSpec((BLKE, D), lambda i: (i, 0)),
            pl.BlockSpec((BLKE, D), lambda i: (i, 0)),
            pl.BlockSpec((BLKE, D), lambda i: (i, 0)),
            pl.BlockSpec((BLKE, 1), lambda i: (i, 0)),
            full((8, 128)), full((G, D)), full((D, D)), full((D, D)),
            full((1, D)),
        ],
        out_specs=[
            pl.BlockSpec((BLKE, D), lambda i: (i, 0)),
            full((G, D)), full((G, D)),
        ],
        out_shape=[
            jax.ShapeDtypeStruct((EP, D), F32),
            jax.ShapeDtypeStruct((G, D), F32),
            jax.ShapeDtypeStruct((G, D), F32),
        ],
    )(ga, gb, e_pad, row_col, starts, ue, w1c, w2, be2)


def _sc_scatter_body(enew, col2d, zeros_hbm, zcol_hbm, ones_hbm,
                     aggp, cntp,
                     buf, idxc, cbuf, onesv, aggS, cntS, lsem, asem):
    s = lax.axis_index("s")
    c = lax.axis_index("c")
    wid = c * NS + s
    rows = NP // NS
    pltpu.sync_copy(ones_hbm, onesv)

    pltpu.sync_copy(zeros_hbm, buf.at[0])
    for t in range(rows // 128):
        pltpu.sync_copy(
            buf.at[0],
            aggS.at[pl.ds(pl.multiple_of(s * rows + t * 128, 128), 128)])
    pltpu.sync_copy(zcol_hbm, cbuf)
    pltpu.sync_copy(cbuf, cntS.at[pl.ds(pl.multiple_of(s * rows, rows), rows)])
    plsc.subcore_barrier()

    base128 = wid * (EPW // 128)
    NB = 2

    def body(k, carry):
        blk = pl.multiple_of(base128 + k * 8, 8)
        pltpu.sync_copy(col2d.at[pl.ds(blk, 8)], idxc)
        lcps = [None] * 8
        acps = [None] * 8
        ccps = [None] * 8

        def start_add(p):
            psl = p % NB
            lcps[p].wait()
            acps[p] = pltpu.async_copy(buf.at[psl], aggS.at[idxc.at[p]],
                                       asem, add=True)
            ccps[p] = pltpu.async_copy(onesv, cntS.at[idxc.at[p]],
                                       asem, add=True)

        for m in range(8):
            sl = m % NB
            if m >= NB:
                acps[m - NB].wait()
                ccps[m - NB].wait()
            off = pl.multiple_of((blk + m) * 128, 128)
            lcps[m] = pltpu.async_copy(enew.at[pl.ds(off, 128)], buf.at[sl],
                                       lsem)
            if m >= 1:
                start_add(m - 1)
        start_add(7)
        for p in range(8 - NB, 8):
            acps[p].wait()
            ccps[p].wait()
        return carry

    lax.fori_loop(0, EPW // 1024, body, 0)
    plsc.subcore_barrier()

    for t in range(rows // 128):
        r = pl.multiple_of(s * rows + t * 128, 128)
        pltpu.sync_copy(aggS.at[pl.ds(r, 128)], buf.at[0])
        pltpu.sync_copy(buf.at[0],
                        aggp.at[pl.ds(pl.multiple_of(c * NP + r, 128), 128)])
    pltpu.sync_copy(cntS.at[pl.ds(pl.multiple_of(s * rows, rows), rows)], cbuf)
    pltpu.sync_copy(
        cbuf, cntp.at[pl.ds(pl.multiple_of(c * NP + s * rows, rows), rows)])


def _sc_scatter(e_new, col2d, zeros_blk, zcol, onesb):
    mesh = plsc.VectorSubcoreMesh(core_axis_name="c", subcore_axis_name="s")
    f = pl.kernel(
        _sc_scatter_body,
        out_type=[
            jax.ShapeDtypeStruct((NC * NP, D), F32),
            jax.ShapeDtypeStruct((NC * NP,), F32),
        ],
        mesh=mesh,
        scratch_types=[
            pltpu.VMEM((2, 128, D), F32),
            pltpu.VMEM((8, 128), jnp.int32),
            pltpu.VMEM((NP // NS,), F32),
            pltpu.VMEM((128,), F32),
            pltpu.VMEM_SHARED((NP, D), F32),
            pltpu.VMEM_SHARED((NP,), F32),
            pltpu.SemaphoreType.DMA,
            pltpu.SemaphoreType.DMA,
        ],
    )
    return f(e_new, col2d, zeros_blk, zcol, onesb)


def _node_body(xn_ref, p0_ref, p1_ref, c0_ref, c1_ref, b_ref, un_ref,
               u_ref, sue_ref, cne_ref, w1b_ref, bn1_ref, wn2_ref, bn2_ref,
               g1u_ref, g1x_ref, g1e_ref, bg1_ref, wg2_ref, bg2_ref,
               out_ref, uout_ref, sx_ref, cn_ref):
    i = pl.program_id(0)

    @pl.when(i == 0)
    def _():
        sx_ref[...] = jnp.zeros_like(sx_ref)
        cn_ref[...] = jnp.zeros_like(cn_ref)

    cnt = jnp.maximum(c0_ref[...] + c1_ref[...], 1.0)
    agg = (p0_ref[...] + p1_ref[...]) / cnt
    b = b_ref[...]
    onehot = (b == lax.broadcasted_iota(jnp.int32, (1, 8), 1)).astype(F32)
    z = (xn_ref[...] +
         jnp.dot(agg, w1b_ref[...], preferred_element_type=F32) +
         jnp.dot(onehot, un_ref[...], preferred_element_type=F32) +
         bn1_ref[...])
    h = jnp.maximum(z, 0.0)
    x_new = jnp.dot(h, wn2_ref[...], preferred_element_type=F32) + bn2_ref[...]
    out_ref[...] = x_new
    sx_ref[...] += lax.dot_general(onehot, x_new, (((0,), (0,)), ((), ())),
                                   preferred_element_type=F32)
    ones = jnp.ones((BLKN, 128), F32)
    cn_ref[...] += lax.dot_general(onehot, ones, (((0,), (0,)), ((), ())),
                                   preferred_element_type=F32)

    @pl.when(i == pl.num_programs(0) - 1)
    def _():
        sxm = sx_ref[...] / jnp.maximum(cn_ref[...], 1.0)
        sem = sue_ref[...] / jnp.maximum(cne_ref[...], 1.0)
        zg = (jnp.dot(u_ref[...], g1u_ref[...], preferred_element_type=F32) +
              jnp.dot(sxm, g1x_ref[...], preferred_element_type=F32) +
              jnp.dot(sem, g1e_ref[...], preferred_element_type=F32) +
              bg1_ref[...])
        hg = jnp.maximum(zg, 0.0)
        uout_ref[...] = jnp.dot(hg, wg2_ref[...], preferred_element_type=F32) + bg2_ref[...]


def _node_global(xn, p0, p1, c0, c1, batch_col, un, u, sue, cne,
                 w1b, bn1, wn2, bn2, g1u, g1x, g1e, bg1, wg2, bg2):
    grid = (NP // BLKN,)
    full = lambda shp: pl.BlockSpec(shp, lambda i: (0, 0))
    return pl.pallas_call(
        _node_body,
        grid=grid,
        in_specs=[
            pl.BlockSpec((BLKN, D), lambda i: (i, 0)),
            pl.BlockSpec((BLKN, D), lambda i: (i, 0)),
            pl.BlockSpec((BLKN, D), lambda i: (i, 0)),
            pl.BlockSpec((BLKN, 1), lambda i: (i, 0)),
            pl.BlockSpec((BLKN, 1), lambda i: (i, 0)),
            pl.BlockSpec((BLKN, 1), lambda i: (i, 0)),
            full((G, D)), full((G, D)), full((G, D)), full((G, D)),
            full((D, D)), full((1, D)), full((D, D)), full((1, D)),
            full((D, D)), full((D, D)), full((D, D)), full((1, D)),
            full((D, D)), full((1, D)),
        ],
        out_specs=[
            pl.BlockSpec((BLKN, D), lambda i: (i, 0)),
            full((G, D)),
        ],
        out_shape=[
            jax.ShapeDtypeStruct((NP, D), F32),
            jax.ShapeDtypeStruct((G, D), F32),
        ],
        scratch_shapes=[
            pltpu.VMEM((G, D), F32),
            pltpu.VMEM((G, D), F32),
        ],
    )(xn, p0, p1, c0, c1, batch_col, un, u, sue, cne,
      w1b, bn1, wn2, bn2, g1u, g1x, g1e, bg1, wg2, bg2)


def kernel(nodes_in, edge_index, edges_in, global_in, batch_index,
           We1, be1, We2, be2, Wn1, bn1, Wn2, bn2, Wg1, bg1, Wg2, bg2):
    x = jnp.pad(nodes_in, ((0, NP - N), (0, 0)))
    batch_col = jnp.pad(batch_index.astype(jnp.int32), (0, NP - N),
                        constant_values=G)[:, None]
    row = jnp.pad(edge_index[0].astype(jnp.int32), (0, EP - E),
                  constant_values=NP - 1)
    col = jnp.pad(edge_index[1].astype(jnp.int32), (0, EP - E),
                  constant_values=NP - 1)
    e_pad = jnp.pad(edges_in, ((0, EP - E), (0, 0)))
    u = global_in

    xa, xb, xn, ue, un, starts = _preproj(
        x, batch_col, u,
        We1[0:128], We1[128:256], Wn1[0:128], We1[384:512],
        be1[None, :], Wn1[256:384])

    row2d = row.reshape(EP // 128, 128)
    col2d = col.reshape(EP // 128, 128)
    ga, gb = _sc_gather(xa, xb, row2d, col2d)

    e_new, sue, cne = _edge_mlp(ga, gb, e_pad, row[:, None], starts, ue,
                                We1[256:384], We2, be2[None, :])

    zeros_blk = jnp.zeros((128, D), F32)
    zcol = jnp.zeros((NP // NS,), F32)
    onesb = jnp.ones((128,), F32)
    aggp, cntp = _sc_scatter(e_new, col2d, zeros_blk, zcol, onesb)
    cntp = cntp[:, None]

    x_new, u_new = _node_global(
        xn, aggp[:NP], aggp[NP:], cntp[:NP], cntp[NP:], batch_col,
        un, u, sue, cne,
        Wn1[128:256], bn1[None, :], Wn2, bn2[None, :],
        Wg1[0:128], Wg1[128:256], Wg1[256:384], bg1[None, :],
        Wg2, bg2[None, :])

    return (x_new[:N], e_new[:E], u_new)


# speedup vs baseline: 4.4786x; 1.0098x over previous
import functools

import jax
import jax.numpy as jnp
from jax import lax
from jax.experimental import pallas as pl
from jax.experimental.pallas import tpu as pltpu
from jax.experimental.pallas import tpu_sc as plsc

D = 128
N, E, G = 10000, 320000, 8
NP, EP = 10240, 327680
NC, NS = 2, 16
NW = NC * NS
EPW = EP // NW
BLKN = 1024
BLKE = 2048
F32 = jnp.float32


def _preproj_body(x_ref, b_ref, u_ref, wa_ref, wb_ref, wn_ref, wud_ref,
                  be1_ref, wun_ref,
                  xa_ref, xb_ref, xn_ref, ue_ref, un_ref, starts_ref):
    i = pl.program_id(0)
    x = x_ref[...]
    xa_ref[...] = jnp.dot(x, wa_ref[...], preferred_element_type=F32)
    xb_ref[...] = jnp.dot(x, wb_ref[...], preferred_element_type=F32)
    xn_ref[...] = jnp.dot(x, wn_ref[...], preferred_element_type=F32)

    @pl.when(i == 0)
    def _():
        u = u_ref[...]
        ue_ref[...] = jnp.dot(u, wud_ref[...], preferred_element_type=F32) + be1_ref[...]
        un_ref[...] = jnp.dot(u, wun_ref[...], preferred_element_type=F32)
        starts_ref[...] = jnp.zeros_like(starts_ref)

    b = b_ref[...]
    g = lax.broadcasted_iota(jnp.int32, (1, 128), 1)
    cmp = (b < g).astype(jnp.int32)
    s = jnp.sum(cmp, axis=0, keepdims=True)
    starts_ref[...] += jnp.broadcast_to(s, (8, 128))


def _preproj(x, batch_col, u, wa, wb, wn, wud, be1, wun):
    grid = (NP // BLKN,)
    full = lambda shp: pl.BlockSpec(shp, lambda i: (0, 0))
    return pl.pallas_call(
        _preproj_body,
        grid=grid,
        in_specs=[
            pl.BlockSpec((BLKN, D), lambda i: (i, 0)),
            pl.BlockSpec((BLKN, 1), lambda i: (i, 0)),
            full((G, D)), full((D, D)), full((D, D)), full((D, D)),
            full((D, D)), full((1, D)), full((D, D)),
        ],
        out_specs=[
            pl.BlockSpec((BLKN, D), lambda i: (i, 0)),
            pl.BlockSpec((BLKN, D), lambda i: (i, 0)),
            pl.BlockSpec((BLKN, D), lambda i: (i, 0)),
            full((G, D)), full((G, D)), full((8, 128)),
        ],
        out_shape=[
            jax.ShapeDtypeStruct((NP, D), F32),
            jax.ShapeDtypeStruct((NP, D), F32),
            jax.ShapeDtypeStruct((NP, D), F32),
            jax.ShapeDtypeStruct((G, D), F32),
            jax.ShapeDtypeStruct((G, D), F32),
            jax.ShapeDtypeStruct((8, 128), jnp.int32),
        ],
    )(x, batch_col, u, wa, wb, wn, wud, be1, wun)


BF16 = jnp.bfloat16
F0, F1 = 14, 6
E0 = F0 * 1024 * NS


def _sc_gather_body(xa_hbm, xb_hbm, row2d, col2d, ga_hbm, gb_hbm,
                    idxr, idxc, bufa, bufb, sem, ssem):
    s = lax.axis_index("s")
    c = lax.axis_index("c")
    base = jnp.where(c == 0, s * (F0 * 1024), E0 + s * (F1 * 1024))
    n_out = jnp.where(c == 0, F0, F1)
    NB = 3
    toff = jnp.zeros((16,), jnp.int32) + c * NP

    def body(k, carry):
        rb = pl.multiple_of(base // 128 + k * 8, 8)
        i0 = pltpu.async_copy(row2d.at[pl.ds(rb, 8)], idxr, sem)
        i1 = pltpu.async_copy(col2d.at[pl.ds(rb, 8)], idxc, sem)
        i0.wait()
        i1.wait()
        for r in range(8):
            for j in range(8):
                sl16 = pl.ds(j * 16, 16)
                idxr[r, sl16] = idxr[r, sl16] + toff
                idxc[r, sl16] = idxc[r, sl16] + toff
        gcps = [None] * 8
        scps = [None] * 8

        def start_store(p):
            psl = p % NB
            gcps[p][0].wait()
            gcps[p][1].wait()
            off = pl.multiple_of(base + k * 1024 + p * 128, 128)
            scps[p] = (
                pltpu.async_copy(bufa.at[psl], ga_hbm.at[pl.ds(off, 128)], ssem),
                pltpu.async_copy(bufb.at[psl], gb_hbm.at[pl.ds(off, 128)], ssem),
            )

        for m in range(8):
            sl = m % NB
            if m >= NB:
                scps[m - NB][0].wait()
                scps[m - NB][1].wait()
            gcps[m] = (
                pltpu.async_copy(xa_hbm.at[idxr.at[m]], bufa.at[sl], sem),
                pltpu.async_copy(xb_hbm.at[idxc.at[m]], bufb.at[sl], sem),
            )
            if m >= 2:
                start_store(m - 2)
        start_store(6)
        start_store(7)
        for p in range(8 - NB, 8):
            scps[p][0].wait()
            scps[p][1].wait()
        return carry

    lax.fori_loop(0, n_out, body, 0)


def _sc_gather(xa, xb, row2d, col2d):
    mesh = plsc.VectorSubcoreMesh(core_axis_name="c", subcore_axis_name="s")
    f = pl.kernel(
        _sc_gather_body,
        out_type=[
            jax.ShapeDtypeStruct((EP, D), F32),
            jax.ShapeDtypeStruct((EP, D), F32),
        ],
        mesh=mesh,
        scratch_types=[
            pltpu.VMEM((8, 128), jnp.int32),
            pltpu.VMEM((8, 128), jnp.int32),
            pltpu.VMEM((3, 128, D), F32),
            pltpu.VMEM((3, 128, D), F32),
            pltpu.SemaphoreType.DMA,
            pltpu.SemaphoreType.DMA,
        ],
    )
    return f(jnp.concatenate([xa, xa], axis=0),
             jnp.concatenate([xb, xb], axis=0), row2d, col2d)


def _edge_body(ga_ref, gb_ref, e_ref, r_ref, s_ref, ue_ref, w1c_ref, w2_ref,
               be2_ref, out_ref, sue_ref, cne_ref):
    i = pl.program_id(0)
    r = r_ref[...]
    gi = i * BLKE + lax.broadcasted_iota(jnp.int32, (BLKE, 1), 0)
    valid = (gi < E).astype(F32)
    bi = jnp.zeros((BLKE, 1), jnp.int32)
    for gg in range(1, 8):
        bi = bi + (r >= s_ref[0:1, gg:gg + 1]).astype(jnp.int32)
    onehot = (bi == lax.broadcasted_iota(jnp.int32, (1, 8), 1)).astype(F32)
    onehot = onehot * valid
    uterm = jnp.dot(onehot, ue_ref[...], preferred_element_type=F32)
    h = jnp.maximum(ga_ref[...].astype(F32) + gb_ref[...].astype(F32) +
                    jnp.dot(e_ref[...], w1c_ref[...], preferred_element_type=F32) +
                    uterm, 0.0)
    e_new = jnp.dot(h, w2_ref[...], preferred_element_type=F32) + be2_ref[...]
    out_ref[...] = e_new

    @pl.when(i == 0)
    def _():
        sue_ref[...] = jnp.zeros_like(sue_ref)
        cne_ref[...] = jnp.zeros_like(cne_ref)
    sue_ref[...] += lax.dot_general(onehot, e_new, (((0,), (0,)), ((), ())),
                                    preferred_element_type=F32)
    ones = jnp.ones((BLKE, 128), F32)
    cne_ref[...] += lax.dot_general(onehot, ones, (((0,), (0,)), ((), ())),
                                    preferred_element_type=F32)


def _edge_mlp(ga, gb, e_pad, row_col, starts, ue, w1c, w2, be2):
    grid = (EP // BLKE,)
    full = lambda shp: pl.BlockSpec(shp, lambda i: (0, 0))
    return pl.pallas_call(
        _edge_body,
        grid=grid,
        in_specs=[
            pl.BlockSpec((BLKE, D), lambda i: (i, 0)),
            pl.BlockSpec((BLKE, D), lambda i: (i, 0)),
            pl.BlockSpec((BLKE, D), lambda i: (i, 0)),
            pl.BlockSpec((BLKE, 1), lambda i: (i, 0)),
            full((8, 128)), full((G, D)), full((D, D)), full((D, D)),
            full((1, D)),
        ],
        out_specs=[
            pl.BlockSpec((BLKE, D), lambda i: (i, 0)),
            full((G, D)), full((G, D)),
        ],
        out_shape=[
            jax.ShapeDtypeStruct((EP, D), F32),
            jax.ShapeDtypeStruct((G, D), F32),
            jax.ShapeDtypeStruct((G, D), F32),
        ],
    )(ga, gb, e_pad, row_col, starts, ue, w1c, w2, be2)


def _sc_scatter_body(enew, col2d, zeros_hbm, zcol_hbm, ones_hbm,
                     aggp, cntp,
                     buf, idxc, cbuf, onesv, aggS, cntS, lsem, asem):
    s = lax.axis_index("s")
    c = lax.axis_index("c")
    wid = c * NS + s
    rows = NP // NS
    pltpu.sync_copy(ones_hbm, onesv)

    pltpu.sync_copy(zeros_hbm, buf.at[0])
    for t in range(rows // 128):
        pltpu.sync_copy(
            buf.at[0],
            aggS.at[pl.ds(pl.multiple_of(s * rows + t * 128, 128), 128)])
    pltpu.sync_copy(zcol_hbm, cbuf)
    pltpu.sync_copy(cbuf, cntS.at[pl.ds(pl.multiple_of(s * rows, rows), rows)])
    plsc.subcore_barrier()

    base128 = wid * (EPW // 128)
    NB = 2

    def body(k, carry):
        blk = pl.multiple_of(base128 + k * 8, 8)
        pltpu.sync_copy(col2d.at[pl.ds(blk, 8)], idxc)
        lcps = [None] * 8
        acps = [None] * 8
        ccps = [None] * 8

        def start_add(p):
            psl = p % NB
            lcps[p].wait()
            acps[p] = pltpu.async_copy(buf.at[psl], aggS.at[idxc.at[p]],
                                       asem, add=True)
            ccps[p] = pltpu.async_copy(onesv, cntS.at[idxc.at[p]],
                                       asem, add=True)

        for m in range(8):
            sl = m % NB
            if m >= NB:
                acps[m - NB].wait()
                ccps[m - NB].wait()
            off = pl.multiple_of((blk + m) * 128, 128)
            lcps[m] = pltpu.async_copy(enew.at[pl.ds(off, 128)], buf.at[sl],
                                       lsem)
            if m >= 1:
                start_add(m - 1)
        start_add(7)
        for p in range(8 - NB, 8):
            acps[p].wait()
            ccps[p].wait()
        return carry

    lax.fori_loop(0, EPW // 1024, body, 0)
    plsc.subcore_barrier()

    for t in range(rows // 128):
        r = pl.multiple_of(s * rows + t * 128, 128)
        pltpu.sync_copy(aggS.at[pl.ds(r, 128)], buf.at[0])
        pltpu.sync_copy(buf.at[0],
                        aggp.at[pl.ds(pl.multiple_of(c * NP + r, 128), 128)])
    pltpu.sync_copy(cntS.at[pl.ds(pl.multiple_of(s * rows, rows), rows)], cbuf)
    pltpu.sync_copy(
        cbuf, cntp.at[pl.ds(pl.multiple_of(c * NP + s * rows, rows), rows)])


def _sc_scatter(e_new, col2d, zeros_blk, zcol, onesb):
    mesh = plsc.VectorSubcoreMesh(core_axis_name="c", subcore_axis_name="s")
    f = pl.kernel(
        _sc_scatter_body,
        out_type=[
            jax.ShapeDtypeStruct((NC * NP, D), F32),
            jax.ShapeDtypeStruct((NC * NP,), F32),
        ],
        mesh=mesh,
        scratch_types=[
            pltpu.VMEM((2, 128, D), F32),
            pltpu.VMEM((8, 128), jnp.int32),
            pltpu.VMEM((NP // NS,), F32),
            pltpu.VMEM((128,), F32),
            pltpu.VMEM_SHARED((NP, D), F32),
            pltpu.VMEM_SHARED((NP,), F32),
            pltpu.SemaphoreType.DMA,
            pltpu.SemaphoreType.DMA,
        ],
    )
    return f(e_new, col2d, zeros_blk, zcol, onesb)


def _node_body(xn_ref, p0_ref, p1_ref, c0_ref, c1_ref, b_ref, un_ref,
               u_ref, sue_ref, cne_ref, w1b_ref, bn1_ref, wn2_ref, bn2_ref,
               g1u_ref, g1x_ref, g1e_ref, bg1_ref, wg2_ref, bg2_ref,
               out_ref, uout_ref, sx_ref, cn_ref):
    i = pl.program_id(0)

    @pl.when(i == 0)
    def _():
        sx_ref[...] = jnp.zeros_like(sx_ref)
        cn_ref[...] = jnp.zeros_like(cn_ref)

    cnt = jnp.maximum(c0_ref[...] + c1_ref[...], 1.0)
    agg = (p0_ref[...] + p1_ref[...]) / cnt
    b = b_ref[...]
    onehot = (b == lax.broadcasted_iota(jnp.int32, (1, 8), 1)).astype(F32)
    z = (xn_ref[...] +
         jnp.dot(agg, w1b_ref[...], preferred_element_type=F32) +
         jnp.dot(onehot, un_ref[...], preferred_element_type=F32) +
         bn1_ref[...])
    h = jnp.maximum(z, 0.0)
    x_new = jnp.dot(h, wn2_ref[...], preferred_element_type=F32) + bn2_ref[...]
    out_ref[...] = x_new
    sx_ref[...] += lax.dot_general(onehot, x_new, (((0,), (0,)), ((), ())),
                                   preferred_element_type=F32)
    ones = jnp.ones((BLKN, 128), F32)
    cn_ref[...] += lax.dot_general(onehot, ones, (((0,), (0,)), ((), ())),
                                   preferred_element_type=F32)

    @pl.when(i == pl.num_programs(0) - 1)
    def _():
        sxm = sx_ref[...] / jnp.maximum(cn_ref[...], 1.0)
        sem = sue_ref[...] / jnp.maximum(cne_ref[...], 1.0)
        zg = (jnp.dot(u_ref[...], g1u_ref[...], preferred_element_type=F32) +
              jnp.dot(sxm, g1x_ref[...], preferred_element_type=F32) +
              jnp.dot(sem, g1e_ref[...], preferred_element_type=F32) +
              bg1_ref[...])
        hg = jnp.maximum(zg, 0.0)
        uout_ref[...] = jnp.dot(hg, wg2_ref[...], preferred_element_type=F32) + bg2_ref[...]


def _node_global(xn, p0, p1, c0, c1, batch_col, un, u, sue, cne,
                 w1b, bn1, wn2, bn2, g1u, g1x, g1e, bg1, wg2, bg2):
    grid = (NP // BLKN,)
    full = lambda shp: pl.BlockSpec(shp, lambda i: (0, 0))
    return pl.pallas_call(
        _node_body,
        grid=grid,
        in_specs=[
            pl.BlockSpec((BLKN, D), lambda i: (i, 0)),
            pl.BlockSpec((BLKN, D), lambda i: (i, 0)),
            pl.BlockSpec((BLKN, D), lambda i: (i, 0)),
            pl.BlockSpec((BLKN, 1), lambda i: (i, 0)),
            pl.BlockSpec((BLKN, 1), lambda i: (i, 0)),
            pl.BlockSpec((BLKN, 1), lambda i: (i, 0)),
            full((G, D)), full((G, D)), full((G, D)), full((G, D)),
            full((D, D)), full((1, D)), full((D, D)), full((1, D)),
            full((D, D)), full((D, D)), full((D, D)), full((1, D)),
            full((D, D)), full((1, D)),
        ],
        out_specs=[
            pl.BlockSpec((BLKN, D), lambda i: (i, 0)),
            full((G, D)),
        ],
        out_shape=[
            jax.ShapeDtypeStruct((NP, D), F32),
            jax.ShapeDtypeStruct((G, D), F32),
        ],
        scratch_shapes=[
            pltpu.VMEM((G, D), F32),
            pltpu.VMEM((G, D), F32),
        ],
    )(xn, p0, p1, c0, c1, batch_col, un, u, sue, cne,
      w1b, bn1, wn2, bn2, g1u, g1x, g1e, bg1, wg2, bg2)


def kernel(nodes_in, edge_index, edges_in, global_in, batch_index,
           We1, be1, We2, be2, Wn1, bn1, Wn2, bn2, Wg1, bg1, Wg2, bg2):
    x = jnp.pad(nodes_in, ((0, NP - N), (0, 0)))
    batch_col = jnp.pad(batch_index.astype(jnp.int32), (0, NP - N),
                        constant_values=G)[:, None]
    row = jnp.pad(edge_index[0].astype(jnp.int32), (0, EP - E),
                  constant_values=NP - 1)
    col = jnp.pad(edge_index[1].astype(jnp.int32), (0, EP - E),
                  constant_values=NP - 1)
    e_pad = jnp.pad(edges_in, ((0, EP - E), (0, 0)))
    u = global_in

    xa, xb, xn, ue, un, starts = _preproj(
        x, batch_col, u,
        We1[0:128], We1[128:256], Wn1[0:128], We1[384:512],
        be1[None, :], Wn1[256:384])

    row2d = row.reshape(EP // 128, 128)
    col2d = col.reshape(EP // 128, 128)
    ga, gb = _sc_gather(xa, xb, row2d, col2d)

    e_new, sue, cne = _edge_mlp(ga, gb, e_pad, row[:, None], starts, ue,
                                We1[256:384], We2, be2[None, :])

    zeros_blk = jnp.zeros((128, D), F32)
    zcol = jnp.zeros((NP // NS,), F32)
    onesb = jnp.ones((128,), F32)
    aggp, cntp = _sc_scatter(e_new, col2d, zeros_blk, zcol, onesb)
    cntp = cntp[:, None]

    x_new, u_new = _node_global(
        xn, aggp[:NP], aggp[NP:], cntp[:NP], cntp[NP:], batch_col,
        un, u, sue, cne,
        Wn1[128:256], bn1[None, :], Wn2, bn2[None, :],
        Wg1[0:128], Wg1[128:256], Wg1[256:384], bg1[None, :],
        Wg2, bg2[None, :])

    return (x_new[:N], e_new[:E], u_new)


# speedup vs baseline: 4.5650x; 1.0193x over previous
import functools

import jax
import jax.numpy as jnp
from jax import lax
from jax.experimental import pallas as pl
from jax.experimental.pallas import tpu as pltpu
from jax.experimental.pallas import tpu_sc as plsc

D = 128
N, E, G = 10000, 320000, 8
NP, EP = 10240, 327680
NC, NS = 2, 16
NW = NC * NS
EPW = EP // NW
BLKN = 1024
BLKE = 2048
F32 = jnp.float32


def _preproj_body(x_ref, b_ref, u_ref, wa_ref, wb_ref, wn_ref, wud_ref,
                  be1_ref, wun_ref,
                  xa_ref, xb_ref, xn_ref, ue_ref, un_ref, starts_ref):
    i = pl.program_id(0)
    x = x_ref[...]
    xa_ref[...] = jnp.dot(x, wa_ref[...], preferred_element_type=F32)
    xb_ref[...] = jnp.dot(x, wb_ref[...], preferred_element_type=F32)
    xn_ref[...] = jnp.dot(x, wn_ref[...], preferred_element_type=F32)

    @pl.when(i == 0)
    def _():
        u = u_ref[...]
        ue_ref[...] = jnp.dot(u, wud_ref[...], preferred_element_type=F32) + be1_ref[...]
        un_ref[...] = jnp.dot(u, wun_ref[...], preferred_element_type=F32)
        starts_ref[...] = jnp.zeros_like(starts_ref)

    b = b_ref[...]
    g = lax.broadcasted_iota(jnp.int32, (1, 128), 1)
    cmp = (b < g).astype(jnp.int32)
    s = jnp.sum(cmp, axis=0, keepdims=True)
    starts_ref[...] += jnp.broadcast_to(s, (8, 128))


def _preproj(x, batch_col, u, wa, wb, wn, wud, be1, wun):
    grid = (NP // BLKN,)
    full = lambda shp: pl.BlockSpec(shp, lambda i: (0, 0))
    return pl.pallas_call(
        _preproj_body,
        grid=grid,
        in_specs=[
            pl.BlockSpec((BLKN, D), lambda i: (i, 0)),
            pl.BlockSpec((BLKN, 1), lambda i: (i, 0)),
            full((G, D)), full((D, D)), full((D, D)), full((D, D)),
            full((D, D)), full((1, D)), full((D, D)),
        ],
        out_specs=[
            pl.BlockSpec((BLKN, D), lambda i: (i, 0)),
            pl.BlockSpec((BLKN, D), lambda i: (i, 0)),
            pl.BlockSpec((BLKN, D), lambda i: (i, 0)),
            full((G, D)), full((G, D)), full((8, 128)),
        ],
        out_shape=[
            jax.ShapeDtypeStruct((NP, D), F32),
            jax.ShapeDtypeStruct((NP, D), F32),
            jax.ShapeDtypeStruct((NP, D), F32),
            jax.ShapeDtypeStruct((G, D), F32),
            jax.ShapeDtypeStruct((G, D), F32),
            jax.ShapeDtypeStruct((8, 128), jnp.int32),
        ],
    )(x, batch_col, u, wa, wb, wn, wud, be1, wun)


BF16 = jnp.bfloat16
F0, F1 = 14, 6
E0 = F0 * 1024 * NS


def _sc_gather_body(xa_hbm, xb_hbm, row2d, col2d, ga_hbm, gb_hbm,
                    idxr, idxc, bufa, bufb, sem, ssem):
    s = lax.axis_index("s")
    c = lax.axis_index("c")
    base = jnp.where(c == 0, s * (F0 * 1024), E0 + s * (F1 * 1024))
    n_out = jnp.where(c == 0, F0, F1)
    NB = 3
    toff = jnp.zeros((16,), jnp.int32) + c * NP

    def body(k, carry):
        rb = pl.multiple_of(base // 128 + k * 8, 8)
        i0 = pltpu.async_copy(row2d.at[pl.ds(rb, 8)], idxr, sem)
        i1 = pltpu.async_copy(col2d.at[pl.ds(rb, 8)], idxc, sem)
        i0.wait()
        i1.wait()
        for r in range(8):
            for j in range(8):
                sl16 = pl.ds(j * 16, 16)
                idxr[r, sl16] = idxr[r, sl16] + toff
                idxc[r, sl16] = idxc[r, sl16] + toff
        gcps = [None] * 8
        scps = [None] * 8

        def start_store(p):
            psl = p % NB
            gcps[p][0].wait()
            gcps[p][1].wait()
            off = pl.multiple_of(base + k * 1024 + p * 128, 128)
            scps[p] = (
                pltpu.async_copy(bufa.at[psl], ga_hbm.at[pl.ds(off, 128)], ssem),
                pltpu.async_copy(bufb.at[psl], gb_hbm.at[pl.ds(off, 128)], ssem),
            )

        for m in range(8):
            sl = m % NB
            if m >= NB:
                scps[m - NB][0].wait()
                scps[m - NB][1].wait()
            gcps[m] = (
                pltpu.async_copy(xa_hbm.at[idxr.at[m]], bufa.at[sl], sem),
                pltpu.async_copy(xb_hbm.at[idxc.at[m]], bufb.at[sl], sem),
            )
            if m >= 1:
                start_store(m - 1)
        start_store(7)
        for p in range(8 - NB, 8):
            scps[p][0].wait()
            scps[p][1].wait()
        return carry

    lax.fori_loop(0, n_out, body, 0)


def _sc_gather(xa, xb, row2d, col2d):
    mesh = plsc.VectorSubcoreMesh(core_axis_name="c", subcore_axis_name="s")
    f = pl.kernel(
        _sc_gather_body,
        out_type=[
            jax.ShapeDtypeStruct((EP, D), F32),
            jax.ShapeDtypeStruct((EP, D), F32),
        ],
        mesh=mesh,
        scratch_types=[
            pltpu.VMEM((8, 128), jnp.int32),
            pltpu.VMEM((8, 128), jnp.int32),
            pltpu.VMEM((3, 128, D), F32),
            pltpu.VMEM((3, 128, D), F32),
            pltpu.SemaphoreType.DMA,
            pltpu.SemaphoreType.DMA,
        ],
    )
    return f(jnp.concatenate([xa, xa], axis=0),
             jnp.concatenate([xb, xb], axis=0), row2d, col2d)


def _edge_body(ga_ref, gb_ref, e_ref, r_ref, s_ref, ue_ref, w1c_ref, w2_ref,
               be2_ref, out_ref, sue_ref, cne_ref):
    i = pl.program_id(0)
    r = r_ref[...]
    gi = i * BLKE + lax.broadcasted_iota(jnp.int32, (BLKE, 1), 0)
    valid = (gi < E).astype(F32)
    bi = jnp.zeros((BLKE, 1), jnp.int32)
    for gg in range(1, 8):
        bi = bi + (r >= s_ref[0:1, gg:gg + 1]).astype(jnp.int32)
    onehot = (bi == lax.broadcasted_iota(jnp.int32, (1, 8), 1)).astype(F32)
    onehot = onehot * valid
    uterm = jnp.dot(onehot, ue_ref[...], preferred_element_type=F32)
    h = jnp.maximum(ga_ref[...].astype(F32) + gb_ref[...].astype(F32) +
                    jnp.dot(e_ref[...], w1c_ref[...], preferred_element_type=F32) +
                    uterm, 0.0)
    e_new = jnp.dot(h, w2_ref[...], preferred_element_type=F32) + be2_ref[...]
    out_ref[...] = e_new

    @pl.when(i == 0)
    def _():
        sue_ref[...] = jnp.zeros_like(sue_ref)
        cne_ref[...] = jnp.zeros_like(cne_ref)
    sue_ref[...] += lax.dot_general(onehot, e_new, (((0,), (0,)), ((), ())),
                                    preferred_element_type=F32)
    ones = jnp.ones((BLKE, 128), F32)
    cne_ref[...] += lax.dot_general(onehot, ones, (((0,), (0,)), ((), ())),
                                    preferred_element_type=F32)


def _edge_mlp(ga, gb, e_pad, row_col, starts, ue, w1c, w2, be2):
    grid = (EP // BLKE,)
    full = lambda shp: pl.BlockSpec(shp, lambda i: (0, 0))
    return pl.pallas_call(
        _edge_body,
        grid=grid,
        in_specs=[
            pl.BlockSpec((BLKE, D), lambda i: (i, 0)),
            pl.BlockSpec((BLKE, D), lambda i: (i, 0)),
            pl.BlockSpec((BLKE, D), lambda i: (i, 0)),
            pl.BlockSpec((BLKE, 1), lambda i: (i, 0)),
            full((8, 128)), full((G, D)), full((D, D)), full((D, D)),
            full((1, D)),
        ],
        out_specs=[
            pl.BlockSpec((BLKE, D), lambda i: (i, 0)),
            full((G, D)), full((G, D)),
        ],
        out_shape=[
            jax.ShapeDtypeStruct((EP, D), F32),
            jax.ShapeDtypeStruct((G, D), F32),
            jax.ShapeDtypeStruct((G, D), F32),
        ],
    )(ga, gb, e_pad, row_col, starts, ue, w1c, w2, be2)


def _sc_scatter_body(enew, col2d, zeros_hbm, zcol_hbm, ones_hbm,
                     aggp, cntp,
                     buf, idxc, cbuf, onesv, aggS, cntS, lsem, asem):
    s = lax.axis_index("s")
    c = lax.axis_index("c")
    wid = c * NS + s
    rows = NP // NS
    pltpu.sync_copy(ones_hbm, onesv)

    pltpu.sync_copy(zeros_hbm, buf.at[0])
    for t in range(rows // 128):
        pltpu.sync_copy(
            buf.at[0],
            aggS.at[pl.ds(pl.multiple_of(s * rows + t * 128, 128), 128)])
    pltpu.sync_copy(zcol_hbm, cbuf)
    pltpu.sync_copy(cbuf, cntS.at[pl.ds(pl.multiple_of(s * rows, rows), rows)])
    plsc.subcore_barrier()

    base128 = wid * (EPW // 128)
    NB = 2

    def body(k, carry):
        blk = pl.multiple_of(base128 + k * 8, 8)
        pltpu.sync_copy(col2d.at[pl.ds(blk, 8)], idxc)
        lcps = [None] * 8
        acps = [None] * 8
        ccps = [None] * 8

        def start_add(p):
            psl = p % NB
            lcps[p].wait()
            acps[p] = pltpu.async_copy(buf.at[psl], aggS.at[idxc.at[p]],
                                       asem, add=True)
            ccps[p] = pltpu.async_copy(onesv, cntS.at[idxc.at[p]],
                                       asem, add=True)

        for m in range(8):
            sl = m % NB
            if m >= NB:
                acps[m - NB].wait()
                ccps[m - NB].wait()
            off = pl.multiple_of((blk + m) * 128, 128)
            lcps[m] = pltpu.async_copy(enew.at[pl.ds(off, 128)], buf.at[sl],
                                       lsem)
            if m >= 1:
                start_add(m - 1)
        start_add(7)
        for p in range(8 - NB, 8):
            acps[p].wait()
            ccps[p].wait()
        return carry

    lax.fori_loop(0, EPW // 1024, body, 0)
    plsc.subcore_barrier()

    for t in range(rows // 128):
        r = pl.multiple_of(s * rows + t * 128, 128)
        pltpu.sync_copy(aggS.at[pl.ds(r, 128)], buf.at[0])
        pltpu.sync_copy(buf.at[0],
                        aggp.at[pl.ds(pl.multiple_of(c * NP + r, 128), 128)])
    pltpu.sync_copy(cntS.at[pl.ds(pl.multiple_of(s * rows, rows), rows)], cbuf)
    pltpu.sync_copy(
        cbuf, cntp.at[pl.ds(pl.multiple_of(c * NP + s * rows, rows), rows)])


def _sc_scatter(e_new, col2d, zeros_blk, zcol, onesb):
    mesh = plsc.VectorSubcoreMesh(core_axis_name="c", subcore_axis_name="s")
    f = pl.kernel(
        _sc_scatter_body,
        out_type=[
            jax.ShapeDtypeStruct((NC * NP, D), F32),
            jax.ShapeDtypeStruct((NC * NP,), F32),
        ],
        mesh=mesh,
        scratch_types=[
            pltpu.VMEM((2, 128, D), F32),
            pltpu.VMEM((8, 128), jnp.int32),
            pltpu.VMEM((NP // NS,), F32),
            pltpu.VMEM((128,), F32),
            pltpu.VMEM_SHARED((NP, D), F32),
            pltpu.VMEM_SHARED((NP,), F32),
            pltpu.SemaphoreType.DMA,
            pltpu.SemaphoreType.DMA,
        ],
    )
    return f(e_new, col2d, zeros_blk, zcol, onesb)


def _node_body(xn_ref, p0_ref, p1_ref, c0_ref, c1_ref, b_ref, un_ref,
               u_ref, sue_ref, cne_ref, w1b_ref, bn1_ref, wn2_ref, bn2_ref,
               g1u_ref, g1x_ref, g1e_ref, bg1_ref, wg2_ref, bg2_ref,
               out_ref, uout_ref, sx_ref, cn_ref):
    i = pl.program_id(0)

    @pl.when(i == 0)
    def _():
        sx_ref[...] = jnp.zeros_like(sx_ref)
        cn_ref[...] = jnp.zeros_like(cn_ref)

    cnt = jnp.maximum(c0_ref[...] + c1_ref[...], 1.0)
    agg = (p0_ref[...] + p1_ref[...]) / cnt
    b = b_ref[...]
    onehot = (b == lax.broadcasted_iota(jnp.int32, (1, 8), 1)).astype(F32)
    z = (xn_ref[...] +
         jnp.dot(agg, w1b_ref[...], preferred_element_type=F32) +
         jnp.dot(onehot, un_ref[...], preferred_element_type=F32) +
         bn1_ref[...])
    h = jnp.maximum(z, 0.0)
    x_new = jnp.dot(h, wn2_ref[...], preferred_element_type=F32) + bn2_ref[...]
    out_ref[...] = x_new
    sx_ref[...] += lax.dot_general(onehot, x_new, (((0,), (0,)), ((), ())),
                                   preferred_element_type=F32)
    ones = jnp.ones((BLKN, 128), F32)
    cn_ref[...] += lax.dot_general(onehot, ones, (((0,), (0,)), ((), ())),
                                   preferred_element_type=F32)

    @pl.when(i == pl.num_programs(0) - 1)
    def _():
        sxm = sx_ref[...] / jnp.maximum(cn_ref[...], 1.0)
        sem = sue_ref[...] / jnp.maximum(cne_ref[...], 1.0)
        zg = (jnp.dot(u_ref[...], g1u_ref[...], preferred_element_type=F32) +
              jnp.dot(sxm, g1x_ref[...], preferred_element_type=F32) +
              jnp.dot(sem, g1e_ref[...], preferred_element_type=F32) +
              bg1_ref[...])
        hg = jnp.maximum(zg, 0.0)
        uout_ref[...] = jnp.dot(hg, wg2_ref[...], preferred_element_type=F32) + bg2_ref[...]


def _node_global(xn, p0, p1, c0, c1, batch_col, un, u, sue, cne,
                 w1b, bn1, wn2, bn2, g1u, g1x, g1e, bg1, wg2, bg2):
    grid = (NP // BLKN,)
    full = lambda shp: pl.BlockSpec(shp, lambda i: (0, 0))
    return pl.pallas_call(
        _node_body,
        grid=grid,
        in_specs=[
            pl.BlockSpec((BLKN, D), lambda i: (i, 0)),
            pl.BlockSpec((BLKN, D), lambda i: (i, 0)),
            pl.BlockSpec((BLKN, D), lambda i: (i, 0)),
            pl.BlockSpec((BLKN, 1), lambda i: (i, 0)),
            pl.BlockSpec((BLKN, 1), lambda i: (i, 0)),
            pl.BlockSpec((BLKN, 1), lambda i: (i, 0)),
            full((G, D)), full((G, D)), full((G, D)), full((G, D)),
            full((D, D)), full((1, D)), full((D, D)), full((1, D)),
            full((D, D)), full((D, D)), full((D, D)), full((1, D)),
            full((D, D)), full((1, D)),
        ],
        out_specs=[
            pl.BlockSpec((BLKN, D), lambda i: (i, 0)),
            full((G, D)),
        ],
        out_shape=[
            jax.ShapeDtypeStruct((NP, D), F32),
            jax.ShapeDtypeStruct((G, D), F32),
        ],
        scratch_shapes=[
            pltpu.VMEM((G, D), F32),
            pltpu.VMEM((G, D), F32),
        ],
    )(xn, p0, p1, c0, c1, batch_col, un, u, sue, cne,
      w1b, bn1, wn2, bn2, g1u, g1x, g1e, bg1, wg2, bg2)


def kernel(nodes_in, edge_index, edges_in, global_in, batch_index,
           We1, be1, We2, be2, Wn1, bn1, Wn2, bn2, Wg1, bg1, Wg2, bg2):
    x = jnp.pad(nodes_in, ((0, NP - N), (0, 0)))
    batch_col = jnp.pad(batch_index.astype(jnp.int32), (0, NP - N),
                        constant_values=G)[:, None]
    row = jnp.pad(edge_index[0].astype(jnp.int32), (0, EP - E),
                  constant_values=NP - 1)
    col = jnp.pad(edge_index[1].astype(jnp.int32), (0, EP - E),
                  constant_values=NP - 1)
    e_pad = jnp.pad(edges_in, ((0, EP - E), (0, 0)))
    u = global_in

    xa, xb, xn, ue, un, starts = _preproj(
        x, batch_col, u,
        We1[0:128], We1[128:256], Wn1[0:128], We1[384:512],
        be1[None, :], Wn1[256:384])

    row2d = row.reshape(EP // 128, 128)
    col2d = col.reshape(EP // 128, 128)
    ga, gb = _sc_gather(xa, xb, row2d, col2d)

    e_new, sue, cne = _edge_mlp(ga, gb, e_pad, row[:, None], starts, ue,
                                We1[256:384], We2, be2[None, :])

    zeros_blk = jnp.zeros((128, D), F32)
    zcol = jnp.zeros((NP // NS,), F32)
    onesb = jnp.ones((128,), F32)
    aggp, cntp = _sc_scatter(e_new, col2d, zeros_blk, zcol, onesb)
    cntp = cntp[:, None]

    x_new, u_new = _node_global(
        xn, aggp[:NP], aggp[NP:], cntp[:NP], cntp[NP:], batch_col,
        un, u, sue, cne,
        Wn1[128:256], bn1[None, :], Wn2, bn2[None, :],
        Wg1[0:128], Wg1[128:256], Wg1[256:384], bg1[None, :],
        Wg2, bg2[None, :])

    return (x_new[:N], e_new[:E], u_new)
